# baseline (speedup 1.0000x reference)
import functools

import jax
import jax.numpy as jnp
from jax import lax
from jax.experimental import pallas as pl
from jax.experimental.pallas import tpu as pltpu
from jax.experimental.pallas import tpu_sc as plsc

N, E, D, H, C, G = 10000, 160000, 256, 256, 16, 64
NP = 10240
NC, NS = 2, 16
NT = NC * NS
EPT = E // NT
EPS = E // NS
GPT = 16
GPN = NP // GPT
RPT = NP // NS
E3 = 163840
DCH = E3 // NT // 128
ACH = E3 // NS // 128

_MESH = plsc.VectorSubcoreMesh(core_axis_name="c", subcore_axis_name="s")


def _degree_body(edges, gids, out_o, out_i, out_c,
                 acc_o, acc_i, acc_c, sb0, sb1, sb2, sb3, db0, db1, db2, db3,
                 gid_a, ones_v, zb, sem):
    c = lax.axis_index("c")
    s = lax.axis_index("s")
    w = s * NC + c

    @pl.when(w < GPT)
    def _():
        pltpu.sync_copy(gids.at[w], gid_a)

    def _fill(i, _):
        zb[pl.ds(i * 16, 16)] = jnp.zeros((16,), jnp.float32)
        return 0
    lax.fori_loop(0, RPT // 16, _fill, 0)
    for i in range(8):
        ones_v[pl.ds(i * 16, 16)] = jnp.ones((16,), jnp.float32)

    pltpu.sync_copy(zb, acc_o.at[pl.ds(s * RPT, RPT)])
    pltpu.sync_copy(zb, acc_i.at[pl.ds(s * RPT, RPT)])

    @pl.when(s == 0)
    def _():
        pltpu.sync_copy(zb.at[pl.ds(0, 128)], acc_c)

    plsc.subcore_barrier()

    sbufs = (sb0, sb1, sb2, sb3)
    dbufs = (db0, db1, db2, db3)

    def _grp(gi, _):
        for k in range(4):
            j = gi * 4 + k
            pltpu.sync_copy(edges.at[0, w, j], sbufs[k])
            pltpu.sync_copy(edges.at[1, w, j], dbufs[k])
        for k in range(4):
            pltpu.async_copy(ones_v, acc_o.at[sbufs[k]], sem, add=True)
            pltpu.async_copy(ones_v, acc_i.at[dbufs[k]], sem, add=True)
        for k in range(4):
            pltpu.make_async_copy(ones_v, acc_o.at[sbufs[0]], sem).wait()
            pltpu.make_async_copy(ones_v, acc_i.at[dbufs[0]], sem).wait()
        return 0
    lax.fori_loop(0, DCH // 4, _grp, 0)

    @pl.when(w < GPT)
    def _():
        def _gfire(j, _):
            pltpu.async_copy(ones_v, acc_c.at[gid_a.at[j]], sem, add=True)
            return 0
        lax.fori_loop(0, GPN // 128, _gfire, 0)

        def _gdrain(j, _):
            pltpu.make_async_copy(ones_v, acc_c.at[gid_a.at[0]], sem).wait()
            return 0
        lax.fori_loop(0, GPN // 128, _gdrain, 0)

    plsc.subcore_barrier()
    pltpu.sync_copy(acc_o.at[pl.ds(s * RPT, RPT)], out_o.at[c, pl.ds(s * RPT, RPT)])
    pltpu.sync_copy(acc_i.at[pl.ds(s * RPT, RPT)], out_i.at[c, pl.ds(s * RPT, RPT)])

    @pl.when(s == 0)
    def _():
        pltpu.sync_copy(acc_c, out_c.at[c])


_degree_kernel = pl.kernel(
    _degree_body,
    out_type=(jax.ShapeDtypeStruct((NC, NP), jnp.float32),
              jax.ShapeDtypeStruct((NC, NP), jnp.float32),
              jax.ShapeDtypeStruct((NC, 128), jnp.float32)),
    mesh=_MESH,
    scratch_types=[
        pltpu.VMEM_SHARED((NP,), jnp.float32),
        pltpu.VMEM_SHARED((NP,), jnp.float32),
        pltpu.VMEM_SHARED((128,), jnp.float32),
        pltpu.VMEM((128,), jnp.int32),
        pltpu.VMEM((128,), jnp.int32),
        pltpu.VMEM((128,), jnp.int32),
        pltpu.VMEM((128,), jnp.int32),
        pltpu.VMEM((128,), jnp.int32),
        pltpu.VMEM((128,), jnp.int32),
        pltpu.VMEM((128,), jnp.int32),
        pltpu.VMEM((128,), jnp.int32),
        pltpu.VMEM((GPN // 128, 128), jnp.int32),
        pltpu.VMEM((128,), jnp.float32),
        pltpu.VMEM((RPT,), jnp.float32),
        pltpu.SemaphoreType.DMA,
    ],
)


def _agg_body(hn, edges, out, acc, sv0, sv1, dv0, dv1, r0, r1, sg0, sg1):
    c = lax.axis_index("c")
    s = lax.axis_index("s")
    off = c * N

    def _zfill(i, _):
        for q in range(8):
            r0[i, pl.ds(q * 16, 16)] = jnp.zeros((16,), jnp.float32)
        return 0
    lax.fori_loop(0, 128, _zfill, 0)
    for t in range(RPT // 128):
        pltpu.sync_copy(r0, acc.at[pl.ds(s * RPT + t * 128, 128)])
    plsc.subcore_barrier()

    sv = (sv0, sv1)
    dv = (dv0, dv1)
    rows = (r0, r1)
    sg = (sg0, sg1)

    def _jj(jj, _):
        descs = [None, None]
        for u in range(2):
            j = 2 * jj + u
            pltpu.sync_copy(edges.at[0, s, pl.ds(j * 128, 128)], sv[u])
            pltpu.sync_copy(edges.at[1, s, pl.ds(j * 128, 128)], dv[u])
            for q in range(8):
                sv[u][pl.ds(q * 16, 16)] = sv[u][pl.ds(q * 16, 16)] + off
            descs[u] = pltpu.async_copy(hn.at[sv[u]], rows[u], sg[u])
        for u in range(2):
            descs[u].wait()
            pltpu.sync_copy(rows[u], acc.at[dv[u]], add=True)
        return 0
    lax.fori_loop(0, ACH // 2, _jj, 0)

    plsc.subcore_barrier()
    for t in range(RPT // 128):
        pltpu.sync_copy(acc.at[pl.ds(s * RPT + t * 128, 128)],
                        out.at[c, pl.ds(s * RPT + t * 128, 128)])


_agg_kernel = pl.kernel(
    _agg_body,
    out_type=jax.ShapeDtypeStruct((NC, NP, 128), jnp.float32),
    mesh=_MESH,
    scratch_types=[
        pltpu.VMEM_SHARED((NP, 128), jnp.float32),
        pltpu.VMEM((128,), jnp.int32),
        pltpu.VMEM((128,), jnp.int32),
        pltpu.VMEM((128,), jnp.int32),
        pltpu.VMEM((128,), jnp.int32),
        pltpu.VMEM((128, 128), jnp.float32),
        pltpu.VMEM((128, 128), jnp.float32),
        pltpu.SemaphoreType.DMA,
        pltpu.SemaphoreType.DMA,
    ],
)


def _pool_body(h2, offs, out_s, out_m, off_v, rb, ob):
    c = lax.axis_index("c")
    s = lax.axis_index("s")
    w = s * NC + c
    pltpu.sync_copy(offs, off_v)

    for gi in range(2):
        g = w * 2 + gi
        ov = off_v[pl.ds(g, 16)]
        start = ov[0]
        end = ov[1]
        start_al = lax.div(start, 8) * 8
        nch = (end - start_al + 7) // 8

        def _chunk(k, carry):
            r0 = start_al + k * 8
            pltpu.sync_copy(h2.at[pl.ds(r0, 8)], rb)
            sums = list(carry[:16])
            maxs = list(carry[16:])
            for r in range(8):
                valid = ((r0 + r) >= start) & ((r0 + r) < end)
                for q in range(16):
                    v = rb[r, pl.ds(q * 16, 16)]
                    sums[q] = sums[q] + jnp.where(valid, v, 0.0)
                    maxs[q] = jnp.maximum(maxs[q], jnp.where(valid, v, -jnp.inf))
            return tuple(sums + maxs)

        init = tuple([jnp.zeros((16,), jnp.float32)] * 16
                     + [jnp.full((16,), -jnp.inf, jnp.float32)] * 16)
        carry = lax.fori_loop(0, nch, _chunk, init)
        for q in range(16):
            ob[pl.ds(q * 16, 16)] = carry[q]
        pltpu.sync_copy(ob, out_s.at[g])
        for q in range(16):
            ob[pl.ds(q * 16, 16)] = carry[16 + q]
        pltpu.sync_copy(ob, out_m.at[g])


_pool_kernel = pl.kernel(
    _pool_body,
    out_type=(jax.ShapeDtypeStruct((G, H), jnp.float32),
              jax.ShapeDtypeStruct((G, H), jnp.float32)),
    mesh=_MESH,
    scratch_types=[
        pltpu.VMEM((128,), jnp.int32),
        pltpu.VMEM((8, H), jnp.float32),
        pltpu.VMEM((H,), jnp.float32),
    ],
)


_B2 = 400


def _scale_body(x_ref, po_ref, pc_ref, hn_ref, off_ref, cnt_ref):
    deg = po_ref[0] + po_ref[1]
    ns = lax.rsqrt(jnp.maximum(deg, 1.0))
    hn_ref[...] = x_ref[...] * ns
    cnt = pc_ref[0] + pc_ref[1]
    cnt_ref[...] = cnt
    row = lax.broadcasted_iota(jnp.int32, (128, 128), 0)
    col = lax.broadcasted_iota(jnp.int32, (128, 128), 1)
    m = (row < col).astype(jnp.float32)
    off = jnp.dot(cnt, m, preferred_element_type=jnp.float32)
    off_ref[...] = off.astype(jnp.int32)


def _scale_call(x, pdeg_out, pcnt):
    return pl.pallas_call(
        _scale_body,
        grid=(50,),
        in_specs=[
            pl.BlockSpec((_B2, 128), lambda i: (lax.rem(i, 25), lax.div(i, 25))),
            pl.BlockSpec((NC, _B2, 1), lambda i: (0, lax.rem(i, 25), 0)),
            pl.BlockSpec((NC, 1, 128), lambda i: (0, 0, 0)),
        ],
        out_specs=[
            pl.BlockSpec((_B2, 128), lambda i: (i, 0)),
            pl.BlockSpec((1, 128), lambda i: (0, 0)),
            pl.BlockSpec((1, 128), lambda i: (0, 0)),
        ],
        out_shape=(jax.ShapeDtypeStruct((2 * N, 128), jnp.float32),
                   jax.ShapeDtypeStruct((1, 128), jnp.int32),
                   jax.ShapeDtypeStruct((1, 128), jnp.float32)),
    )(x, pdeg_out, pcnt)


_BC = 512


def _conv_body(scale_src, a_ref, di_ref, do_ref, w_ref, b_ref, g_ref, be_ref,
               out_ref):
    di = di_ref[0] + di_ref[1]
    nd = lax.rsqrt(jnp.maximum(di, 1.0))
    a0 = a_ref[0] * nd
    a1 = a_ref[1] * nd
    z = (jnp.dot(a0, w_ref[:128, :], preferred_element_type=jnp.float32)
         + jnp.dot(a1, w_ref[128:, :], preferred_element_type=jnp.float32)
         + b_ref[...])
    mu = jnp.mean(z, axis=-1, keepdims=True)
    var = jnp.mean((z - mu) ** 2, axis=-1, keepdims=True)
    h = (z - mu) / jnp.sqrt(var + 1e-5) * g_ref[...] + be_ref[...]
    h = jnp.maximum(h, 0.0)
    if scale_src:
        do = do_ref[0] + do_ref[1]
        h = h * lax.rsqrt(jnp.maximum(do, 1.0))
    out_ref[...] = h


def _conv_call(scale_src, agg, pdeg_in, pdeg_out, W, b, g, be):
    return pl.pallas_call(
        functools.partial(_conv_body, scale_src),
        grid=(NP // _BC,),
        in_specs=[
            pl.BlockSpec((NC, _BC, 128), lambda i: (0, i, 0)),
            pl.BlockSpec((NC, _BC, 1), lambda i: (0, i, 0)),
            pl.BlockSpec((NC, _BC, 1), lambda i: (0, i, 0)),
            pl.BlockSpec((H, H), lambda i: (0, 0)),
            pl.BlockSpec((1, H), lambda i: (0, 0)),
            pl.BlockSpec((1, H), lambda i: (0, 0)),
            pl.BlockSpec((1, H), lambda i: (0, 0)),
        ],
        out_specs=pl.BlockSpec((_BC, H), lambda i: (i, 0)),
        out_shape=jax.ShapeDtypeStruct((NP, H), jnp.float32),
    )(agg, pdeg_in, pdeg_out, W, b, g, be)


def _head_body(hs_ref, hx_ref, cnt_ref, w1_ref, b1_ref, w2_ref, b2_ref,
               w3_ref, b3_ref, g3_ref, be3_ref, g4_ref, be4_ref, out_ref):
    row = lax.broadcasted_iota(jnp.int32, (G, 128), 0)
    col = lax.broadcasted_iota(jnp.int32, (G, 128), 1)
    m = (row == col).astype(jnp.float32)
    cc = lax.dot_general(m, cnt_ref[...], (((1,), (1,)), ((), ())),
                         preferred_element_type=jnp.float32)
    mean = hs_ref[...] / jnp.maximum(cc, 1.0)

    def _l2n(v):
        n = jnp.sqrt(jnp.sum(v * v, axis=-1, keepdims=True))
        return v / jnp.maximum(n, 1e-12)

    def _ln(z, gg, bb):
        mu = jnp.mean(z, axis=-1, keepdims=True)
        var = jnp.mean((z - mu) ** 2, axis=-1, keepdims=True)
        return (z - mu) / jnp.sqrt(var + 1e-5) * gg + bb

    hm = _l2n(mean)
    hx = _l2n(hx_ref[...])
    z = (jnp.dot(hm, w1_ref[:H, :], preferred_element_type=jnp.float32)
         + jnp.dot(hx, w1_ref[H:, :], preferred_element_type=jnp.float32)
         + b1_ref[...])
    z = jnp.maximum(_ln(z, g3_ref[...], be3_ref[...]), 0.0)
    z = jnp.dot(z, w2_ref[...], preferred_element_type=jnp.float32) + b2_ref[...]
    z = jnp.maximum(_ln(z, g4_ref[...], be4_ref[...]), 0.0)
    out_ref[...] = (jnp.dot(z, w3_ref[...], preferred_element_type=jnp.float32)
                    + b3_ref[...])


def _head_call(hg_sum, hg_max, cnt, w1, b1, w2, b2, w3, b3, g3, be3, g4, be4):
    return pl.pallas_call(
        _head_body,
        out_shape=jax.ShapeDtypeStruct((G, C), jnp.float32),
    )(hg_sum, hg_max, cnt, w1, b1, w2, b2, w3, b3, g3, be3, g4, be4)


def kernel(x, edge_index, graph_ids, conv1_W, conv1_b, conv2_W, conv2_b,
           ln1_g, ln1_b, ln2_g, ln2_b, ln3_g, ln3_b, ln4_g, ln4_b,
           cls1_W, cls1_b, cls2_W, cls2_b, cls3_W, cls3_b):
    ei = edge_index.astype(jnp.int32)
    pad_d = jnp.full((2, E3 - E), N, jnp.int32)
    er_deg = jnp.concatenate([ei, pad_d], axis=1).reshape(2, NT, DCH, 128)
    pad_a = jnp.stack([jnp.zeros((E3 - E,), jnp.int32),
                       jnp.full((E3 - E,), N, jnp.int32)])
    er_agg = jnp.concatenate([ei, pad_a], axis=1).reshape(2, NS, NS * 640)
    gr = jnp.concatenate([graph_ids.astype(jnp.int32),
                          jnp.full((NP - N,), G, jnp.int32)]).reshape(
        GPT, GPN // 128, 128)

    pdeg_out, pdeg_in, pcnt = _degree_kernel(er_deg, gr)
    pdeg_out = pdeg_out.reshape(NC, NP, 1)
    pdeg_in = pdeg_in.reshape(NC, NP, 1)
    pcnt = pcnt.reshape(NC, 1, 128)

    hn_s, offsets, cnt = _scale_call(x, pdeg_out, pcnt)

    agg1 = _agg_kernel(hn_s, er_agg)
    h1n = _conv_call(True, agg1, pdeg_in, pdeg_out, conv1_W,
                     conv1_b.reshape(1, H), ln1_g.reshape(1, H),
                     ln1_b.reshape(1, H))
    hn2_s = jnp.concatenate([h1n[:N, :128], h1n[:N, 128:]], axis=0)

    agg2 = _agg_kernel(hn2_s, er_agg)
    h2 = _conv_call(False, agg2, pdeg_in, pdeg_out, conv2_W,
                    conv2_b.reshape(1, H), ln2_g.reshape(1, H),
                    ln2_b.reshape(1, H))

    hg_sum, hg_max = _pool_kernel(h2, offsets.reshape(128))

    return _head_call(hg_sum, hg_max, cnt, cls1_W, cls1_b.reshape(1, H),
                      cls2_W, cls2_b.reshape(1, H), cls3_W,
                      cls3_b.reshape(1, C), ln3_g.reshape(1, H),
                      ln3_b.reshape(1, H), ln4_g.reshape(1, H),
                      ln4_b.reshape(1, H))


# speedup vs baseline: 1.3770x; 1.3770x over previous
import functools

import jax
import jax.numpy as jnp
from jax import lax
from jax.experimental import pallas as pl
from jax.experimental.pallas import tpu as pltpu
from jax.experimental.pallas import tpu_sc as plsc

N, E, D, H, C, G = 10000, 160000, 256, 256, 16, 64
NP = 10240
NC, NS = 2, 16
NT = NC * NS
EPT = E // NT
EPS = E // NS
GPT = 16
GPN = NP // GPT
RPT = NP // NS
E3 = 163840
DCH = E3 // NT // 128
ACH = E3 // NS // 128

_MESH = plsc.VectorSubcoreMesh(core_axis_name="c", subcore_axis_name="s")


def _degree_body(edges, gids, out_o, out_i, out_c,
                 acc_o, acc_i, acc_c, sb0, sb1, sb2, sb3, db0, db1, db2, db3,
                 gid_a, ones_v, zb, sem):
    c = lax.axis_index("c")
    s = lax.axis_index("s")
    w = s * NC + c

    @pl.when(w < GPT)
    def _():
        pltpu.sync_copy(gids.at[w], gid_a)

    def _fill(i, _):
        zb[pl.ds(i * 16, 16)] = jnp.zeros((16,), jnp.float32)
        return 0
    lax.fori_loop(0, RPT // 16, _fill, 0)
    for i in range(8):
        ones_v[pl.ds(i * 16, 16)] = jnp.ones((16,), jnp.float32)

    pltpu.sync_copy(zb, acc_o.at[pl.ds(s * RPT, RPT)])
    pltpu.sync_copy(zb, acc_i.at[pl.ds(s * RPT, RPT)])

    @pl.when(s == 0)
    def _():
        pltpu.sync_copy(zb.at[pl.ds(0, 128)], acc_c)

    plsc.subcore_barrier()

    sbufs = (sb0, sb1, sb2, sb3)
    dbufs = (db0, db1, db2, db3)

    def _grp(gi, _):
        for k in range(4):
            j = gi * 4 + k
            pltpu.sync_copy(edges.at[0, w, j], sbufs[k])
            pltpu.sync_copy(edges.at[1, w, j], dbufs[k])
        for k in range(4):
            pltpu.async_copy(ones_v, acc_o.at[sbufs[k]], sem, add=True)
            pltpu.async_copy(ones_v, acc_i.at[dbufs[k]], sem, add=True)
        for k in range(4):
            pltpu.make_async_copy(ones_v, acc_o.at[sbufs[0]], sem).wait()
            pltpu.make_async_copy(ones_v, acc_i.at[dbufs[0]], sem).wait()
        return 0
    lax.fori_loop(0, DCH // 4, _grp, 0)

    @pl.when(w < GPT)
    def _():
        def _gfire(j, _):
            pltpu.async_copy(ones_v, acc_c.at[gid_a.at[j]], sem, add=True)
            return 0
        lax.fori_loop(0, GPN // 128, _gfire, 0)

        def _gdrain(j, _):
            pltpu.make_async_copy(ones_v, acc_c.at[gid_a.at[0]], sem).wait()
            return 0
        lax.fori_loop(0, GPN // 128, _gdrain, 0)

    plsc.subcore_barrier()
    pltpu.sync_copy(acc_o.at[pl.ds(s * RPT, RPT)], out_o.at[c, pl.ds(s * RPT, RPT)])
    pltpu.sync_copy(acc_i.at[pl.ds(s * RPT, RPT)], out_i.at[c, pl.ds(s * RPT, RPT)])

    @pl.when(s == 0)
    def _():
        pltpu.sync_copy(acc_c, out_c.at[c])


_degree_kernel = pl.kernel(
    _degree_body,
    out_type=(jax.ShapeDtypeStruct((NC, NP), jnp.float32),
              jax.ShapeDtypeStruct((NC, NP), jnp.float32),
              jax.ShapeDtypeStruct((NC, 128), jnp.float32)),
    mesh=_MESH,
    scratch_types=[
        pltpu.VMEM_SHARED((NP,), jnp.float32),
        pltpu.VMEM_SHARED((NP,), jnp.float32),
        pltpu.VMEM_SHARED((128,), jnp.float32),
        pltpu.VMEM((128,), jnp.int32),
        pltpu.VMEM((128,), jnp.int32),
        pltpu.VMEM((128,), jnp.int32),
        pltpu.VMEM((128,), jnp.int32),
        pltpu.VMEM((128,), jnp.int32),
        pltpu.VMEM((128,), jnp.int32),
        pltpu.VMEM((128,), jnp.int32),
        pltpu.VMEM((128,), jnp.int32),
        pltpu.VMEM((GPN // 128, 128), jnp.int32),
        pltpu.VMEM((128,), jnp.float32),
        pltpu.VMEM((RPT,), jnp.float32),
        pltpu.SemaphoreType.DMA,
    ],
)


def _agg_body(hn, edges, out, acc, src_v, dst_v, src16, dst16, rows, sem):
    c = lax.axis_index("c")
    s = lax.axis_index("s")
    off = c * N

    def _zfill(i, _):
        for q in range(8):
            rows[i, pl.ds(q * 16, 16)] = jnp.zeros((16,), jnp.float32)
        return 0
    lax.fori_loop(0, 128, _zfill, 0)
    for t in range(RPT // 128):
        pltpu.sync_copy(rows, acc.at[pl.ds(s * RPT + t * 128, 128)])
    plsc.subcore_barrier()

    def _chunk(j, _):
        pltpu.sync_copy(edges.at[0, s, pl.ds(j * 128, 128)], src_v)
        pltpu.sync_copy(edges.at[1, s, pl.ds(j * 128, 128)], dst_v)
        for q in range(8):
            src_v[pl.ds(q * 16, 16)] = src_v[pl.ds(q * 16, 16)] + off
        pltpu.async_copy(hn.at[src_v], rows, sem).wait()
        pltpu.sync_copy(rows, acc.at[dst_v], add=True)
        return 0
    lax.fori_loop(0, EPS // 128, _chunk, 0)
    tail = (EPS // 128) * 128
    pltpu.sync_copy(edges.at[0, s, pl.ds(tail, 16)], src16)
    pltpu.sync_copy(edges.at[1, s, pl.ds(tail, 16)], dst16)
    src16[pl.ds(0, 16)] = src16[pl.ds(0, 16)] + off
    pltpu.async_copy(hn.at[src16], rows.at[pl.ds(0, 16)], sem).wait()
    pltpu.sync_copy(rows.at[pl.ds(0, 16)], acc.at[dst16], add=True)

    plsc.subcore_barrier()
    for t in range(RPT // 128):
        pltpu.sync_copy(acc.at[pl.ds(s * RPT + t * 128, 128)],
                        out.at[c, pl.ds(s * RPT + t * 128, 128)])


_agg_kernel = pl.kernel(
    _agg_body,
    out_type=jax.ShapeDtypeStruct((NC, NP, 128), jnp.float32),
    mesh=_MESH,
    scratch_types=[
        pltpu.VMEM_SHARED((NP, 128), jnp.float32),
        pltpu.VMEM((128,), jnp.int32),
        pltpu.VMEM((128,), jnp.int32),
        pltpu.VMEM((16,), jnp.int32),
        pltpu.VMEM((16,), jnp.int32),
        pltpu.VMEM((128, 128), jnp.float32),
        pltpu.SemaphoreType.DMA,
    ],
)


def _pool_body(h2, offs, out_s, out_m, off_v, rb, ob):
    c = lax.axis_index("c")
    s = lax.axis_index("s")
    w = s * NC + c
    pltpu.sync_copy(offs, off_v)

    for gi in range(2):
        g = w * 2 + gi
        ov = off_v[pl.ds(g, 16)]
        start = ov[0]
        end = ov[1]
        start_al = lax.div(start, 8) * 8
        nch = (end - start_al + 7) // 8

        def _chunk(k, carry):
            r0 = start_al + k * 8
            pltpu.sync_copy(h2.at[pl.ds(r0, 8)], rb)
            sums = list(carry[:16])
            maxs = list(carry[16:])
            for r in range(8):
                valid = ((r0 + r) >= start) & ((r0 + r) < end)
                for q in range(16):
                    v = rb[r, pl.ds(q * 16, 16)]
                    sums[q] = sums[q] + jnp.where(valid, v, 0.0)
                    maxs[q] = jnp.maximum(maxs[q], jnp.where(valid, v, -jnp.inf))
            return tuple(sums + maxs)

        init = tuple([jnp.zeros((16,), jnp.float32)] * 16
                     + [jnp.full((16,), -jnp.inf, jnp.float32)] * 16)
        carry = lax.fori_loop(0, nch, _chunk, init)
        for q in range(16):
            ob[pl.ds(q * 16, 16)] = carry[q]
        pltpu.sync_copy(ob, out_s.at[g])
        for q in range(16):
            ob[pl.ds(q * 16, 16)] = carry[16 + q]
        pltpu.sync_copy(ob, out_m.at[g])


_pool_kernel = pl.kernel(
    _pool_body,
    out_type=(jax.ShapeDtypeStruct((G, H), jnp.float32),
              jax.ShapeDtypeStruct((G, H), jnp.float32)),
    mesh=_MESH,
    scratch_types=[
        pltpu.VMEM((128,), jnp.int32),
        pltpu.VMEM((8, H), jnp.float32),
        pltpu.VMEM((H,), jnp.float32),
    ],
)


_B2 = 400


def _scale_body(x_ref, po_ref, pc_ref, hn_ref, off_ref, cnt_ref):
    deg = po_ref[0] + po_ref[1]
    ns = lax.rsqrt(jnp.maximum(deg, 1.0))
    hn_ref[...] = x_ref[...] * ns
    cnt = pc_ref[0] + pc_ref[1]
    cnt_ref[...] = cnt
    row = lax.broadcasted_iota(jnp.int32, (128, 128), 0)
    col = lax.broadcasted_iota(jnp.int32, (128, 128), 1)
    m = (row < col).astype(jnp.float32)
    off = jnp.dot(cnt, m, preferred_element_type=jnp.float32)
    off_ref[...] = off.astype(jnp.int32)


def _scale_call(x, pdeg_out, pcnt):
    return pl.pallas_call(
        _scale_body,
        grid=(50,),
        in_specs=[
            pl.BlockSpec((_B2, 128), lambda i: (lax.rem(i, 25), lax.div(i, 25))),
            pl.BlockSpec((NC, _B2, 1), lambda i: (0, lax.rem(i, 25), 0)),
            pl.BlockSpec((NC, 1, 128), lambda i: (0, 0, 0)),
        ],
        out_specs=[
            pl.BlockSpec((_B2, 128), lambda i: (i, 0)),
            pl.BlockSpec((1, 128), lambda i: (0, 0)),
            pl.BlockSpec((1, 128), lambda i: (0, 0)),
        ],
        out_shape=(jax.ShapeDtypeStruct((2 * N, 128), jnp.float32),
                   jax.ShapeDtypeStruct((1, 128), jnp.int32),
                   jax.ShapeDtypeStruct((1, 128), jnp.float32)),
    )(x, pdeg_out, pcnt)


_BC = 512


def _conv_body(scale_src, a_ref, di_ref, do_ref, w_ref, b_ref, g_ref, be_ref,
               out_ref):
    di = di_ref[0] + di_ref[1]
    nd = lax.rsqrt(jnp.maximum(di, 1.0))
    a0 = a_ref[0] * nd
    a1 = a_ref[1] * nd
    z = (jnp.dot(a0, w_ref[:128, :], preferred_element_type=jnp.float32)
         + jnp.dot(a1, w_ref[128:, :], preferred_element_type=jnp.float32)
         + b_ref[...])
    mu = jnp.mean(z, axis=-1, keepdims=True)
    var = jnp.mean((z - mu) ** 2, axis=-1, keepdims=True)
    h = (z - mu) / jnp.sqrt(var + 1e-5) * g_ref[...] + be_ref[...]
    h = jnp.maximum(h, 0.0)
    if scale_src:
        do = do_ref[0] + do_ref[1]
        h = h * lax.rsqrt(jnp.maximum(do, 1.0))
    out_ref[...] = h


def _conv_call(scale_src, agg, pdeg_in, pdeg_out, W, b, g, be):
    return pl.pallas_call(
        functools.partial(_conv_body, scale_src),
        grid=(NP // _BC,),
        in_specs=[
            pl.BlockSpec((NC, _BC, 128), lambda i: (0, i, 0)),
            pl.BlockSpec((NC, _BC, 1), lambda i: (0, i, 0)),
            pl.BlockSpec((NC, _BC, 1), lambda i: (0, i, 0)),
            pl.BlockSpec((H, H), lambda i: (0, 0)),
            pl.BlockSpec((1, H), lambda i: (0, 0)),
            pl.BlockSpec((1, H), lambda i: (0, 0)),
            pl.BlockSpec((1, H), lambda i: (0, 0)),
        ],
        out_specs=pl.BlockSpec((_BC, H), lambda i: (i, 0)),
        out_shape=jax.ShapeDtypeStruct((NP, H), jnp.float32),
    )(agg, pdeg_in, pdeg_out, W, b, g, be)


def _head_body(hs_ref, hx_ref, cnt_ref, w1_ref, b1_ref, w2_ref, b2_ref,
               w3_ref, b3_ref, g3_ref, be3_ref, g4_ref, be4_ref, out_ref):
    row = lax.broadcasted_iota(jnp.int32, (G, 128), 0)
    col = lax.broadcasted_iota(jnp.int32, (G, 128), 1)
    m = (row == col).astype(jnp.float32)
    cc = lax.dot_general(m, cnt_ref[...], (((1,), (1,)), ((), ())),
                         preferred_element_type=jnp.float32)
    mean = hs_ref[...] / jnp.maximum(cc, 1.0)

    def _l2n(v):
        n = jnp.sqrt(jnp.sum(v * v, axis=-1, keepdims=True))
        return v / jnp.maximum(n, 1e-12)

    def _ln(z, gg, bb):
        mu = jnp.mean(z, axis=-1, keepdims=True)
        var = jnp.mean((z - mu) ** 2, axis=-1, keepdims=True)
        return (z - mu) / jnp.sqrt(var + 1e-5) * gg + bb

    hm = _l2n(mean)
    hx = _l2n(hx_ref[...])
    z = (jnp.dot(hm, w1_ref[:H, :], preferred_element_type=jnp.float32)
         + jnp.dot(hx, w1_ref[H:, :], preferred_element_type=jnp.float32)
         + b1_ref[...])
    z = jnp.maximum(_ln(z, g3_ref[...], be3_ref[...]), 0.0)
    z = jnp.dot(z, w2_ref[...], preferred_element_type=jnp.float32) + b2_ref[...]
    z = jnp.maximum(_ln(z, g4_ref[...], be4_ref[...]), 0.0)
    out_ref[...] = (jnp.dot(z, w3_ref[...], preferred_element_type=jnp.float32)
                    + b3_ref[...])


def _head_call(hg_sum, hg_max, cnt, w1, b1, w2, b2, w3, b3, g3, be3, g4, be4):
    return pl.pallas_call(
        _head_body,
        out_shape=jax.ShapeDtypeStruct((G, C), jnp.float32),
    )(hg_sum, hg_max, cnt, w1, b1, w2, b2, w3, b3, g3, be3, g4, be4)


def kernel(x, edge_index, graph_ids, conv1_W, conv1_b, conv2_W, conv2_b,
           ln1_g, ln1_b, ln2_g, ln2_b, ln3_g, ln3_b, ln4_g, ln4_b,
           cls1_W, cls1_b, cls2_W, cls2_b, cls3_W, cls3_b):
    ei = edge_index.astype(jnp.int32)
    pad_d = jnp.full((2, E3 - E), N, jnp.int32)
    er_deg = jnp.concatenate([ei, pad_d], axis=1).reshape(2, NT, DCH, 128)
    er_agg = ei.reshape(2, NS, EPS)
    gr = jnp.concatenate([graph_ids.astype(jnp.int32),
                          jnp.full((NP - N,), G, jnp.int32)]).reshape(
        GPT, GPN // 128, 128)

    pdeg_out, pdeg_in, pcnt = _degree_kernel(er_deg, gr)
    pdeg_out = pdeg_out.reshape(NC, NP, 1)
    pdeg_in = pdeg_in.reshape(NC, NP, 1)
    pcnt = pcnt.reshape(NC, 1, 128)

    hn_s, offsets, cnt = _scale_call(x, pdeg_out, pcnt)

    agg1 = _agg_kernel(hn_s, er_agg)
    h1n = _conv_call(True, agg1, pdeg_in, pdeg_out, conv1_W,
                     conv1_b.reshape(1, H), ln1_g.reshape(1, H),
                     ln1_b.reshape(1, H))
    hn2_s = jnp.concatenate([h1n[:N, :128], h1n[:N, 128:]], axis=0)

    agg2 = _agg_kernel(hn2_s, er_agg)
    h2 = _conv_call(False, agg2, pdeg_in, pdeg_out, conv2_W,
                    conv2_b.reshape(1, H), ln2_g.reshape(1, H),
                    ln2_b.reshape(1, H))

    hg_sum, hg_max = _pool_kernel(h2, offsets.reshape(128))

    return _head_call(hg_sum, hg_max, cnt, cls1_W, cls1_b.reshape(1, H),
                      cls2_W, cls2_b.reshape(1, H), cls3_W,
                      cls3_b.reshape(1, C), ln3_g.reshape(1, H),
                      ln3_b.reshape(1, H), ln4_g.reshape(1, H),
                      ln4_b.reshape(1, H))


# speedup vs baseline: 1.4641x; 1.0633x over previous
import functools

import jax
import jax.numpy as jnp
from jax import lax
from jax.experimental import pallas as pl
from jax.experimental.pallas import tpu as pltpu
from jax.experimental.pallas import tpu_sc as plsc

N, E, D, H, C, G = 10000, 160000, 256, 256, 16, 64
NP = 10240
NC, NS = 2, 16
NT = NC * NS
EPT = E // NT
EPS = E // NS
GPT = 16
GPN = NP // GPT
RPT = NP // NS
E3 = 163840
DCH = E3 // NT // 128
ACH = E3 // NS // 128

_MESH = plsc.VectorSubcoreMesh(core_axis_name="c", subcore_axis_name="s")


def _degree_body(edges, gids, out_o, out_i, out_c,
                 acc_o, acc_i, acc_c, sb0, sb1, sb2, sb3, db0, db1, db2, db3,
                 gid_a, ones_v, zb, sem):
    c = lax.axis_index("c")
    s = lax.axis_index("s")
    w = s * NC + c

    @pl.when(w < GPT)
    def _():
        pltpu.sync_copy(gids.at[w], gid_a)

    def _fill(i, _):
        zb[pl.ds(i * 16, 16)] = jnp.zeros((16,), jnp.float32)
        return 0
    lax.fori_loop(0, RPT // 16, _fill, 0)
    for i in range(8):
        ones_v[pl.ds(i * 16, 16)] = jnp.ones((16,), jnp.float32)

    pltpu.sync_copy(zb, acc_o.at[pl.ds(s * RPT, RPT)])
    pltpu.sync_copy(zb, acc_i.at[pl.ds(s * RPT, RPT)])

    @pl.when(s == 0)
    def _():
        pltpu.sync_copy(zb.at[pl.ds(0, 128)], acc_c)

    plsc.subcore_barrier()

    sbufs = (sb0, sb1, sb2, sb3)
    dbufs = (db0, db1, db2, db3)

    def _grp(gi, _):
        for k in range(4):
            j = gi * 4 + k
            pltpu.sync_copy(edges.at[0, w, j], sbufs[k])
            pltpu.sync_copy(edges.at[1, w, j], dbufs[k])
        for k in range(4):
            pltpu.async_copy(ones_v, acc_o.at[sbufs[k]], sem, add=True)
            pltpu.async_copy(ones_v, acc_i.at[dbufs[k]], sem, add=True)
        for k in range(4):
            pltpu.make_async_copy(ones_v, acc_o.at[sbufs[0]], sem).wait()
            pltpu.make_async_copy(ones_v, acc_i.at[dbufs[0]], sem).wait()
        return 0
    lax.fori_loop(0, DCH // 4, _grp, 0)

    @pl.when(w < GPT)
    def _():
        def _gfire(j, _):
            pltpu.async_copy(ones_v, acc_c.at[gid_a.at[j]], sem, add=True)
            return 0
        lax.fori_loop(0, GPN // 128, _gfire, 0)

        def _gdrain(j, _):
            pltpu.make_async_copy(ones_v, acc_c.at[gid_a.at[0]], sem).wait()
            return 0
        lax.fori_loop(0, GPN // 128, _gdrain, 0)

    plsc.subcore_barrier()
    pltpu.sync_copy(acc_o.at[pl.ds(s * RPT, RPT)], out_o.at[c, pl.ds(s * RPT, RPT)])
    pltpu.sync_copy(acc_i.at[pl.ds(s * RPT, RPT)], out_i.at[c, pl.ds(s * RPT, RPT)])

    @pl.when(s == 0)
    def _():
        pltpu.sync_copy(acc_c, out_c.at[c])


_degree_kernel = pl.kernel(
    _degree_body,
    out_type=(jax.ShapeDtypeStruct((NC, NP), jnp.float32),
              jax.ShapeDtypeStruct((NC, NP), jnp.float32),
              jax.ShapeDtypeStruct((NC, 128), jnp.float32)),
    mesh=_MESH,
    scratch_types=[
        pltpu.VMEM_SHARED((NP,), jnp.float32),
        pltpu.VMEM_SHARED((NP,), jnp.float32),
        pltpu.VMEM_SHARED((128,), jnp.float32),
        pltpu.VMEM((128,), jnp.int32),
        pltpu.VMEM((128,), jnp.int32),
        pltpu.VMEM((128,), jnp.int32),
        pltpu.VMEM((128,), jnp.int32),
        pltpu.VMEM((128,), jnp.int32),
        pltpu.VMEM((128,), jnp.int32),
        pltpu.VMEM((128,), jnp.int32),
        pltpu.VMEM((128,), jnp.int32),
        pltpu.VMEM((GPN // 128, 128), jnp.int32),
        pltpu.VMEM((128,), jnp.float32),
        pltpu.VMEM((RPT,), jnp.float32),
        pltpu.SemaphoreType.DMA,
    ],
)


def _agg_body(hn_a, hn_b, edges, out, acc, src_v, dst_v, src16, dst16,
              rows, sem):
    c = lax.axis_index("c")
    s = lax.axis_index("s")

    def _zfill(i, _):
        for q in range(8):
            rows[i, pl.ds(q * 16, 16)] = jnp.zeros((16,), jnp.float32)
        return 0
    lax.fori_loop(0, 128, _zfill, 0)
    for t in range(RPT // 128):
        pltpu.sync_copy(rows, acc.at[pl.ds(s * RPT + t * 128, 128)])
    plsc.subcore_barrier()

    def _run(hn):
        def _chunk(j, _):
            pltpu.sync_copy(edges.at[0, s, pl.ds(j * 128, 128)], src_v)
            pltpu.sync_copy(edges.at[1, s, pl.ds(j * 128, 128)], dst_v)
            pltpu.async_copy(hn.at[src_v], rows, sem).wait()
            pltpu.sync_copy(rows, acc.at[dst_v], add=True)
            return 0
        lax.fori_loop(0, EPS // 128, _chunk, 0)
        tail = (EPS // 128) * 128
        pltpu.sync_copy(edges.at[0, s, pl.ds(tail, 16)], src16)
        pltpu.sync_copy(edges.at[1, s, pl.ds(tail, 16)], dst16)
        pltpu.async_copy(hn.at[src16], rows.at[pl.ds(0, 16)], sem).wait()
        pltpu.sync_copy(rows.at[pl.ds(0, 16)], acc.at[dst16], add=True)

    @pl.when(c == 0)
    def _():
        _run(hn_a)

    @pl.when(c == 1)
    def _():
        _run(hn_b)

    plsc.subcore_barrier()
    for t in range(RPT // 128):
        pltpu.sync_copy(acc.at[pl.ds(s * RPT + t * 128, 128)],
                        out.at[c, pl.ds(s * RPT + t * 128, 128)])


_agg_kernel = pl.kernel(
    _agg_body,
    out_type=jax.ShapeDtypeStruct((NC, NP, 128), jnp.float32),
    mesh=_MESH,
    scratch_types=[
        pltpu.VMEM_SHARED((NP, 128), jnp.float32),
        pltpu.VMEM((128,), jnp.int32),
        pltpu.VMEM((128,), jnp.int32),
        pltpu.VMEM((16,), jnp.int32),
        pltpu.VMEM((16,), jnp.int32),
        pltpu.VMEM((128, 128), jnp.float32),
        pltpu.SemaphoreType.DMA,
    ],
)


def _pool_body(h2, offs, out_s, out_m, off_v, rb, ob):
    c = lax.axis_index("c")
    s = lax.axis_index("s")
    w = s * NC + c
    pltpu.sync_copy(offs, off_v)

    for gi in range(2):
        g = w * 2 + gi
        ov = off_v[pl.ds(g, 16)]
        start = ov[0]
        end = ov[1]
        start_al = lax.div(start, 8) * 8
        nch = (end - start_al + 15) // 16

        def _chunk(k, carry):
            r0 = start_al + k * 16
            pltpu.sync_copy(h2.at[pl.ds(r0, 16)], rb)
            sums = list(carry[:16])
            maxs = list(carry[16:])
            for r in range(16):
                valid = ((r0 + r) >= start) & ((r0 + r) < end)
                for q in range(16):
                    v = rb[r, pl.ds(q * 16, 16)]
                    sums[q] = sums[q] + jnp.where(valid, v, 0.0)
                    maxs[q] = jnp.maximum(maxs[q], jnp.where(valid, v, -jnp.inf))
            return tuple(sums + maxs)

        init = tuple([jnp.zeros((16,), jnp.float32)] * 16
                     + [jnp.full((16,), -jnp.inf, jnp.float32)] * 16)
        carry = lax.fori_loop(0, nch, _chunk, init)
        for q in range(16):
            ob[pl.ds(q * 16, 16)] = carry[q]
        pltpu.sync_copy(ob, out_s.at[g])
        for q in range(16):
            ob[pl.ds(q * 16, 16)] = carry[16 + q]
        pltpu.sync_copy(ob, out_m.at[g])


_pool_kernel = pl.kernel(
    _pool_body,
    out_type=(jax.ShapeDtypeStruct((G, H), jnp.float32),
              jax.ShapeDtypeStruct((G, H), jnp.float32)),
    mesh=_MESH,
    scratch_types=[
        pltpu.VMEM((128,), jnp.int32),
        pltpu.VMEM((16, H), jnp.float32),
        pltpu.VMEM((H,), jnp.float32),
    ],
)


_B2 = 400


def _scale_body(x_ref, po_ref, pc_ref, hna_ref, hnb_ref, off_ref, cnt_ref):
    deg = po_ref[0] + po_ref[1]
    ns = lax.rsqrt(jnp.maximum(deg, 1.0))
    xb = x_ref[...]
    hna_ref[...] = xb[:, :128] * ns
    hnb_ref[...] = xb[:, 128:] * ns
    cnt = pc_ref[0] + pc_ref[1]
    cnt_ref[...] = cnt
    row = lax.broadcasted_iota(jnp.int32, (128, 128), 0)
    col = lax.broadcasted_iota(jnp.int32, (128, 128), 1)
    m = (row < col).astype(jnp.float32)
    off = jnp.dot(cnt, m, preferred_element_type=jnp.float32)
    off_ref[...] = off.astype(jnp.int32)


def _scale_call(x, pdeg_out, pcnt):
    return pl.pallas_call(
        _scale_body,
        grid=(25,),
        in_specs=[
            pl.BlockSpec((_B2, 256), lambda i: (i, 0)),
            pl.BlockSpec((NC, _B2, 1), lambda i: (0, i, 0)),
            pl.BlockSpec((NC, 1, 128), lambda i: (0, 0, 0)),
        ],
        out_specs=[
            pl.BlockSpec((_B2, 128), lambda i: (i, 0)),
            pl.BlockSpec((_B2, 128), lambda i: (i, 0)),
            pl.BlockSpec((1, 128), lambda i: (0, 0)),
            pl.BlockSpec((1, 128), lambda i: (0, 0)),
        ],
        out_shape=(jax.ShapeDtypeStruct((N, 128), jnp.float32),
                   jax.ShapeDtypeStruct((N, 128), jnp.float32),
                   jax.ShapeDtypeStruct((1, 128), jnp.int32),
                   jax.ShapeDtypeStruct((1, 128), jnp.float32)),
    )(x, pdeg_out, pcnt)


_BC = 512


def _conv_body(split_out, scale_src, a_ref, di_ref, do_ref, w_ref, b_ref,
               g_ref, be_ref, *out_refs):
    di = di_ref[0] + di_ref[1]
    nd = lax.rsqrt(jnp.maximum(di, 1.0))
    a0 = a_ref[0] * nd
    a1 = a_ref[1] * nd
    z = (jnp.dot(a0, w_ref[:128, :], preferred_element_type=jnp.float32)
         + jnp.dot(a1, w_ref[128:, :], preferred_element_type=jnp.float32)
         + b_ref[...])
    mu = jnp.mean(z, axis=-1, keepdims=True)
    var = jnp.mean((z - mu) ** 2, axis=-1, keepdims=True)
    h = (z - mu) / jnp.sqrt(var + 1e-5) * g_ref[...] + be_ref[...]
    h = jnp.maximum(h, 0.0)
    if scale_src:
        do = do_ref[0] + do_ref[1]
        h = h * lax.rsqrt(jnp.maximum(do, 1.0))
    if split_out:
        out_refs[0][...] = h[:, :128]
        out_refs[1][...] = h[:, 128:]
    else:
        out_refs[0][...] = h


def _conv_call(split_out, scale_src, agg, pdeg_in, pdeg_out, W, b, g, be):
    if split_out:
        out_specs = [pl.BlockSpec((_BC, 128), lambda i: (i, 0)),
                     pl.BlockSpec((_BC, 128), lambda i: (i, 0))]
        out_shape = (jax.ShapeDtypeStruct((NP, 128), jnp.float32),
                     jax.ShapeDtypeStruct((NP, 128), jnp.float32))
    else:
        out_specs = pl.BlockSpec((_BC, H), lambda i: (i, 0))
        out_shape = jax.ShapeDtypeStruct((NP, H), jnp.float32)
    return pl.pallas_call(
        functools.partial(_conv_body, split_out, scale_src),
        grid=(NP // _BC,),
        in_specs=[
            pl.BlockSpec((NC, _BC, 128), lambda i: (0, i, 0)),
            pl.BlockSpec((NC, _BC, 1), lambda i: (0, i, 0)),
            pl.BlockSpec((NC, _BC, 1), lambda i: (0, i, 0)),
            pl.BlockSpec((H, H), lambda i: (0, 0)),
            pl.BlockSpec((1, H), lambda i: (0, 0)),
            pl.BlockSpec((1, H), lambda i: (0, 0)),
            pl.BlockSpec((1, H), lambda i: (0, 0)),
        ],
        out_specs=out_specs,
        out_shape=out_shape,
    )(agg, pdeg_in, pdeg_out, W, b, g, be)


def _head_body(hs_ref, hx_ref, cnt_ref, w1_ref, b1_ref, w2_ref, b2_ref,
               w3_ref, b3_ref, g3_ref, be3_ref, g4_ref, be4_ref, out_ref):
    row = lax.broadcasted_iota(jnp.int32, (G, 128), 0)
    col = lax.broadcasted_iota(jnp.int32, (G, 128), 1)
    m = (row == col).astype(jnp.float32)
    cc = lax.dot_general(m, cnt_ref[...], (((1,), (1,)), ((), ())),
                         preferred_element_type=jnp.float32)
    mean = hs_ref[...] / jnp.maximum(cc, 1.0)

    def _l2n(v):
        n = jnp.sqrt(jnp.sum(v * v, axis=-1, keepdims=True))
        return v / jnp.maximum(n, 1e-12)

    def _ln(z, gg, bb):
        mu = jnp.mean(z, axis=-1, keepdims=True)
        var = jnp.mean((z - mu) ** 2, axis=-1, keepdims=True)
        return (z - mu) / jnp.sqrt(var + 1e-5) * gg + bb

    hm = _l2n(mean)
    hx = _l2n(hx_ref[...])
    z = (jnp.dot(hm, w1_ref[:H, :], preferred_element_type=jnp.float32)
         + jnp.dot(hx, w1_ref[H:, :], preferred_element_type=jnp.float32)
         + b1_ref[...])
    z = jnp.maximum(_ln(z, g3_ref[...], be3_ref[...]), 0.0)
    z = jnp.dot(z, w2_ref[...], preferred_element_type=jnp.float32) + b2_ref[...]
    z = jnp.maximum(_ln(z, g4_ref[...], be4_ref[...]), 0.0)
    out_ref[...] = (jnp.dot(z, w3_ref[...], preferred_element_type=jnp.float32)
                    + b3_ref[...])


def _head_call(hg_sum, hg_max, cnt, w1, b1, w2, b2, w3, b3, g3, be3, g4, be4):
    return pl.pallas_call(
        _head_body,
        out_shape=jax.ShapeDtypeStruct((G, C), jnp.float32),
    )(hg_sum, hg_max, cnt, w1, b1, w2, b2, w3, b3, g3, be3, g4, be4)


def kernel(x, edge_index, graph_ids, conv1_W, conv1_b, conv2_W, conv2_b,
           ln1_g, ln1_b, ln2_g, ln2_b, ln3_g, ln3_b, ln4_g, ln4_b,
           cls1_W, cls1_b, cls2_W, cls2_b, cls3_W, cls3_b):
    ei = edge_index.astype(jnp.int32)
    pad_d = jnp.full((2, E3 - E), N, jnp.int32)
    er_deg = jnp.concatenate([ei, pad_d], axis=1).reshape(2, NT, DCH, 128)
    er_agg = ei.reshape(2, NS, EPS)
    gr = jnp.concatenate([graph_ids.astype(jnp.int32),
                          jnp.full((NP - N,), G, jnp.int32)]).reshape(
        GPT, GPN // 128, 128)

    pdeg_out, pdeg_in, pcnt = _degree_kernel(er_deg, gr)
    pdeg_out = pdeg_out.reshape(NC, NP, 1)
    pdeg_in = pdeg_in.reshape(NC, NP, 1)
    pcnt = pcnt.reshape(NC, 1, 128)

    hn_a, hn_b, offsets, cnt = _scale_call(x, pdeg_out, pcnt)

    agg1 = _agg_kernel(hn_a, hn_b, er_agg)
    hn2_a, hn2_b = _conv_call(True, True, agg1, pdeg_in, pdeg_out, conv1_W,
                              conv1_b.reshape(1, H), ln1_g.reshape(1, H),
                              ln1_b.reshape(1, H))

    agg2 = _agg_kernel(hn2_a, hn2_b, er_agg)
    h2 = _conv_call(False, False, agg2, pdeg_in, pdeg_out, conv2_W,
                    conv2_b.reshape(1, H), ln2_g.reshape(1, H),
                    ln2_b.reshape(1, H))

    hg_sum, hg_max = _pool_kernel(h2, offsets.reshape(128))

    return _head_call(hg_sum, hg_max, cnt, cls1_W, cls1_b.reshape(1, H),
                      cls2_W, cls2_b.reshape(1, H), cls3_W,
                      cls3_b.reshape(1, C), ln3_g.reshape(1, H),
                      ln3_b.reshape(1, H), ln4_g.reshape(1, H),
                      ln4_b.reshape(1, H))


# speedup vs baseline: 1.6420x; 1.1215x over previous
import functools

import jax
import jax.numpy as jnp
from jax import lax
from jax.experimental import pallas as pl
from jax.experimental.pallas import tpu as pltpu
from jax.experimental.pallas import tpu_sc as plsc

N, E, D, H, C, G = 10000, 160000, 256, 256, 16, 64
NP = 10240
NC, NS = 2, 16
NT = NC * NS
EPT = E // NT
EPS = E // NS
GPT = 16
GPN = NP // GPT
RPT = NP // NS
E3 = 163840
DCH = E3 // NT // 128
ACH = E3 // NS // 128

_MESH = plsc.VectorSubcoreMesh(core_axis_name="c", subcore_axis_name="s")


def _degree_body(edges, gids, out_o, out_i, out_c,
                 acc_o, acc_i, acc_c, sb0, sb1, sb2, sb3, db0, db1, db2, db3,
                 gid_a, ones_v, zb, sem):
    c = lax.axis_index("c")
    s = lax.axis_index("s")
    w = s * NC + c

    @pl.when(w < GPT)
    def _():
        pltpu.sync_copy(gids.at[w], gid_a)

    def _fill(i, _):
        zb[pl.ds(i * 16, 16)] = jnp.zeros((16,), jnp.float32)
        return 0
    lax.fori_loop(0, RPT // 16, _fill, 0)
    for i in range(8):
        ones_v[pl.ds(i * 16, 16)] = jnp.ones((16,), jnp.float32)

    pltpu.sync_copy(zb, acc_o.at[pl.ds(s * RPT, RPT)])
    pltpu.sync_copy(zb, acc_i.at[pl.ds(s * RPT, RPT)])

    @pl.when(s == 0)
    def _():
        pltpu.sync_copy(zb.at[pl.ds(0, 128)], acc_c)

    plsc.subcore_barrier()

    sbufs = (sb0, sb1, sb2, sb3)
    dbufs = (db0, db1, db2, db3)

    def _grp(gi, _):
        for k in range(4):
            j = gi * 4 + k
            pltpu.sync_copy(edges.at[0, w, j], sbufs[k])
            pltpu.sync_copy(edges.at[1, w, j], dbufs[k])
        for k in range(4):
            pltpu.async_copy(ones_v, acc_o.at[sbufs[k]], sem, add=True)
            pltpu.async_copy(ones_v, acc_i.at[dbufs[k]], sem, add=True)
        for k in range(4):
            pltpu.make_async_copy(ones_v, acc_o.at[sbufs[0]], sem).wait()
            pltpu.make_async_copy(ones_v, acc_i.at[dbufs[0]], sem).wait()
        return 0
    lax.fori_loop(0, DCH // 4, _grp, 0)

    @pl.when(w < GPT)
    def _():
        def _gfire(j, _):
            pltpu.async_copy(ones_v, acc_c.at[gid_a.at[j]], sem, add=True)
            return 0
        lax.fori_loop(0, GPN // 128, _gfire, 0)

        def _gdrain(j, _):
            pltpu.make_async_copy(ones_v, acc_c.at[gid_a.at[0]], sem).wait()
            return 0
        lax.fori_loop(0, GPN // 128, _gdrain, 0)

    plsc.subcore_barrier()
    pltpu.sync_copy(acc_o.at[pl.ds(s * RPT, RPT)], out_o.at[c, pl.ds(s * RPT, RPT)])
    pltpu.sync_copy(acc_i.at[pl.ds(s * RPT, RPT)], out_i.at[c, pl.ds(s * RPT, RPT)])

    @pl.when(s == 0)
    def _():
        pltpu.sync_copy(acc_c, out_c.at[c])


_degree_kernel = pl.kernel(
    _degree_body,
    out_type=(jax.ShapeDtypeStruct((NC, NP), jnp.float32),
              jax.ShapeDtypeStruct((NC, NP), jnp.float32),
              jax.ShapeDtypeStruct((NC, 128), jnp.float32)),
    mesh=_MESH,
    scratch_types=[
        pltpu.VMEM_SHARED((NP,), jnp.float32),
        pltpu.VMEM_SHARED((NP,), jnp.float32),
        pltpu.VMEM_SHARED((128,), jnp.float32),
        pltpu.VMEM((128,), jnp.int32),
        pltpu.VMEM((128,), jnp.int32),
        pltpu.VMEM((128,), jnp.int32),
        pltpu.VMEM((128,), jnp.int32),
        pltpu.VMEM((128,), jnp.int32),
        pltpu.VMEM((128,), jnp.int32),
        pltpu.VMEM((128,), jnp.int32),
        pltpu.VMEM((128,), jnp.int32),
        pltpu.VMEM((GPN // 128, 128), jnp.int32),
        pltpu.VMEM((128,), jnp.float32),
        pltpu.VMEM((RPT,), jnp.float32),
        pltpu.SemaphoreType.DMA,
    ],
)


def _agg_body(hn_a, hn_b, edges, out, acc, src_v, dst_v, src16, dst16,
              rows, sem, se1, se2):
    c = lax.axis_index("c")
    s = lax.axis_index("s")

    def _zfill(i, _):
        for q in range(8):
            rows[i, pl.ds(q * 16, 16)] = jnp.zeros((16,), jnp.float32)
        return 0
    lax.fori_loop(0, 128, _zfill, 0)
    for t in range(RPT // 128):
        pltpu.sync_copy(rows, acc.at[pl.ds(s * RPT + t * 128, 128)])
    plsc.subcore_barrier()

    def _run(hn):
        def _chunk(j, _):
            d1 = pltpu.async_copy(edges.at[0, s, pl.ds(j * 128, 128)],
                                  src_v, se1)
            d2 = pltpu.async_copy(edges.at[1, s, pl.ds(j * 128, 128)],
                                  dst_v, se2)
            d1.wait()
            dg = pltpu.async_copy(hn.at[src_v], rows, sem)
            d2.wait()
            dg.wait()
            pltpu.sync_copy(rows, acc.at[dst_v], add=True)
            return 0
        lax.fori_loop(0, EPS // 128, _chunk, 0)
        tail = (EPS // 128) * 128
        pltpu.sync_copy(edges.at[0, s, pl.ds(tail, 16)], src16)
        pltpu.sync_copy(edges.at[1, s, pl.ds(tail, 16)], dst16)
        pltpu.async_copy(hn.at[src16], rows.at[pl.ds(0, 16)], sem).wait()
        pltpu.sync_copy(rows.at[pl.ds(0, 16)], acc.at[dst16], add=True)

    @pl.when(c == 0)
    def _():
        _run(hn_a)

    @pl.when(c == 1)
    def _():
        _run(hn_b)

    plsc.subcore_barrier()
    for t in range(RPT // 128):
        pltpu.sync_copy(acc.at[pl.ds(s * RPT + t * 128, 128)],
                        out.at[c, pl.ds(s * RPT + t * 128, 128)])


_agg_kernel = pl.kernel(
    _agg_body,
    out_type=jax.ShapeDtypeStruct((NC, NP, 128), jnp.float32),
    mesh=_MESH,
    scratch_types=[
        pltpu.VMEM_SHARED((NP, 128), jnp.float32),
        pltpu.VMEM((128,), jnp.int32),
        pltpu.VMEM((128,), jnp.int32),
        pltpu.VMEM((16,), jnp.int32),
        pltpu.VMEM((16,), jnp.int32),
        pltpu.VMEM((128, 128), jnp.float32),
        pltpu.SemaphoreType.DMA,
        pltpu.SemaphoreType.DMA,
        pltpu.SemaphoreType.DMA,
    ],
)


def _pool_body(h2, offs, out_s, out_m, off_v, rb, ob):
    c = lax.axis_index("c")
    s = lax.axis_index("s")
    w = s * NC + c
    pltpu.sync_copy(offs, off_v)

    for gi in range(2):
        g = w * 2 + gi
        ov = off_v[pl.ds(g, 16)]
        start = ov[0]
        end = ov[1]
        start_al = lax.div(start, 8) * 8
        nch = (end - start_al + 15) // 16

        def _chunk(k, carry):
            r0 = start_al + k * 16
            pltpu.sync_copy(h2.at[pl.ds(r0, 16)], rb)
            sums = list(carry[:16])
            maxs = list(carry[16:])
            for r in range(16):
                valid = ((r0 + r) >= start) & ((r0 + r) < end)
                for q in range(16):
                    v = rb[r, pl.ds(q * 16, 16)]
                    sums[q] = sums[q] + jnp.where(valid, v, 0.0)
                    maxs[q] = jnp.maximum(maxs[q], jnp.where(valid, v, -jnp.inf))
            return tuple(sums + maxs)

        init = tuple([jnp.zeros((16,), jnp.float32)] * 16
                     + [jnp.full((16,), -jnp.inf, jnp.float32)] * 16)
        carry = lax.fori_loop(0, nch, _chunk, init)
        for q in range(16):
            ob[pl.ds(q * 16, 16)] = carry[q]
        pltpu.sync_copy(ob, out_s.at[g])
        for q in range(16):
            ob[pl.ds(q * 16, 16)] = carry[16 + q]
        pltpu.sync_copy(ob, out_m.at[g])


_pool_kernel = pl.kernel(
    _pool_body,
    out_type=(jax.ShapeDtypeStruct((G, H), jnp.float32),
              jax.ShapeDtypeStruct((G, H), jnp.float32)),
    mesh=_MESH,
    scratch_types=[
        pltpu.VMEM((128,), jnp.int32),
        pltpu.VMEM((16, H), jnp.float32),
        pltpu.VMEM((H,), jnp.float32),
    ],
)


_B2 = 400


def _scale_body(x_ref, po_ref, pc_ref, hna_ref, hnb_ref, off_ref, cnt_ref):
    deg = po_ref[0] + po_ref[1]
    ns = lax.rsqrt(jnp.maximum(deg, 1.0))
    xb = x_ref[...]
    hna_ref[...] = xb[:, :128] * ns
    hnb_ref[...] = xb[:, 128:] * ns
    cnt = pc_ref[0] + pc_ref[1]
    cnt_ref[...] = cnt
    row = lax.broadcasted_iota(jnp.int32, (128, 128), 0)
    col = lax.broadcasted_iota(jnp.int32, (128, 128), 1)
    m = (row < col).astype(jnp.float32)
    off = jnp.dot(cnt, m, preferred_element_type=jnp.float32)
    off_ref[...] = off.astype(jnp.int32)


def _scale_call(x, pdeg_out, pcnt):
    return pl.pallas_call(
        _scale_body,
        grid=(25,),
        in_specs=[
            pl.BlockSpec((_B2, 256), lambda i: (i, 0)),
            pl.BlockSpec((NC, _B2, 1), lambda i: (0, i, 0)),
            pl.BlockSpec((NC, 1, 128), lambda i: (0, 0, 0)),
        ],
        out_specs=[
            pl.BlockSpec((_B2, 128), lambda i: (i, 0)),
            pl.BlockSpec((_B2, 128), lambda i: (i, 0)),
            pl.BlockSpec((1, 128), lambda i: (0, 0)),
            pl.BlockSpec((1, 128), lambda i: (0, 0)),
        ],
        out_shape=(jax.ShapeDtypeStruct((N, 128), jnp.float32),
                   jax.ShapeDtypeStruct((N, 128), jnp.float32),
                   jax.ShapeDtypeStruct((1, 128), jnp.int32),
                   jax.ShapeDtypeStruct((1, 128), jnp.float32)),
    )(x, pdeg_out, pcnt)


_BC = 512


def _conv_body(split_out, scale_src, a_ref, di_ref, do_ref, w_ref, b_ref,
               g_ref, be_ref, *out_refs):
    di = di_ref[0] + di_ref[1]
    nd = lax.rsqrt(jnp.maximum(di, 1.0))
    a0 = a_ref[0] * nd
    a1 = a_ref[1] * nd
    z = (jnp.dot(a0, w_ref[:128, :], preferred_element_type=jnp.float32)
         + jnp.dot(a1, w_ref[128:, :], preferred_element_type=jnp.float32)
         + b_ref[...])
    mu = jnp.mean(z, axis=-1, keepdims=True)
    var = jnp.mean((z - mu) ** 2, axis=-1, keepdims=True)
    h = (z - mu) / jnp.sqrt(var + 1e-5) * g_ref[...] + be_ref[...]
    h = jnp.maximum(h, 0.0)
    if scale_src:
        do = do_ref[0] + do_ref[1]
        h = h * lax.rsqrt(jnp.maximum(do, 1.0))
    if split_out:
        out_refs[0][...] = h[:, :128]
        out_refs[1][...] = h[:, 128:]
    else:
        out_refs[0][...] = h


def _conv_call(split_out, scale_src, agg, pdeg_in, pdeg_out, W, b, g, be):
    if split_out:
        out_specs = [pl.BlockSpec((_BC, 128), lambda i: (i, 0)),
                     pl.BlockSpec((_BC, 128), lambda i: (i, 0))]
        out_shape = (jax.ShapeDtypeStruct((NP, 128), jnp.float32),
                     jax.ShapeDtypeStruct((NP, 128), jnp.float32))
    else:
        out_specs = pl.BlockSpec((_BC, H), lambda i: (i, 0))
        out_shape = jax.ShapeDtypeStruct((NP, H), jnp.float32)
    return pl.pallas_call(
        functools.partial(_conv_body, split_out, scale_src),
        grid=(NP // _BC,),
        in_specs=[
            pl.BlockSpec((NC, _BC, 128), lambda i: (0, i, 0)),
            pl.BlockSpec((NC, _BC, 1), lambda i: (0, i, 0)),
            pl.BlockSpec((NC, _BC, 1), lambda i: (0, i, 0)),
            pl.BlockSpec((H, H), lambda i: (0, 0)),
            pl.BlockSpec((1, H), lambda i: (0, 0)),
            pl.BlockSpec((1, H), lambda i: (0, 0)),
            pl.BlockSpec((1, H), lambda i: (0, 0)),
        ],
        out_specs=out_specs,
        out_shape=out_shape,
    )(agg, pdeg_in, pdeg_out, W, b, g, be)


def _head_body(hs_ref, hx_ref, cnt_ref, w1_ref, b1_ref, w2_ref, b2_ref,
               w3_ref, b3_ref, g3_ref, be3_ref, g4_ref, be4_ref, out_ref):
    row = lax.broadcasted_iota(jnp.int32, (G, 128), 0)
    col = lax.broadcasted_iota(jnp.int32, (G, 128), 1)
    m = (row == col).astype(jnp.float32)
    cc = lax.dot_general(m, cnt_ref[...], (((1,), (1,)), ((), ())),
                         preferred_element_type=jnp.float32)
    mean = hs_ref[...] / jnp.maximum(cc, 1.0)

    def _l2n(v):
        n = jnp.sqrt(jnp.sum(v * v, axis=-1, keepdims=True))
        return v / jnp.maximum(n, 1e-12)

    def _ln(z, gg, bb):
        mu = jnp.mean(z, axis=-1, keepdims=True)
        var = jnp.mean((z - mu) ** 2, axis=-1, keepdims=True)
        return (z - mu) / jnp.sqrt(var + 1e-5) * gg + bb

    hm = _l2n(mean)
    hx = _l2n(hx_ref[...])
    z = (jnp.dot(hm, w1_ref[:H, :], preferred_element_type=jnp.float32)
         + jnp.dot(hx, w1_ref[H:, :], preferred_element_type=jnp.float32)
         + b1_ref[...])
    z = jnp.maximum(_ln(z, g3_ref[...], be3_ref[...]), 0.0)
    z = jnp.dot(z, w2_ref[...], preferred_element_type=jnp.float32) + b2_ref[...]
    z = jnp.maximum(_ln(z, g4_ref[...], be4_ref[...]), 0.0)
    out_ref[...] = (jnp.dot(z, w3_ref[...], preferred_element_type=jnp.float32)
                    + b3_ref[...])


def _head_call(hg_sum, hg_max, cnt, w1, b1, w2, b2, w3, b3, g3, be3, g4, be4):
    return pl.pallas_call(
        _head_body,
        out_shape=jax.ShapeDtypeStruct((G, C), jnp.float32),
    )(hg_sum, hg_max, cnt, w1, b1, w2, b2, w3, b3, g3, be3, g4, be4)


def kernel(x, edge_index, graph_ids, conv1_W, conv1_b, conv2_W, conv2_b,
           ln1_g, ln1_b, ln2_g, ln2_b, ln3_g, ln3_b, ln4_g, ln4_b,
           cls1_W, cls1_b, cls2_W, cls2_b, cls3_W, cls3_b):
    ei = edge_index.astype(jnp.int32)
    pad_d = jnp.full((2, E3 - E), N, jnp.int32)
    er_deg = jnp.concatenate([ei, pad_d], axis=1).reshape(2, NT, DCH, 128)
    er_agg = ei.reshape(2, NS, EPS)
    gr = jnp.concatenate([graph_ids.astype(jnp.int32),
                          jnp.full((NP - N,), G, jnp.int32)]).reshape(
        GPT, GPN // 128, 128)

    pdeg_out, pdeg_in, pcnt = _degree_kernel(er_deg, gr)
    pdeg_out = pdeg_out.reshape(NC, NP, 1)
    pdeg_in = pdeg_in.reshape(NC, NP, 1)
    pcnt = pcnt.reshape(NC, 1, 128)

    hn_a, hn_b, offsets, cnt = _scale_call(x, pdeg_out, pcnt)

    agg1 = _agg_kernel(hn_a, hn_b, er_agg)
    hn2_a, hn2_b = _conv_call(True, True, agg1, pdeg_in, pdeg_out, conv1_W,
                              conv1_b.reshape(1, H), ln1_g.reshape(1, H),
                              ln1_b.reshape(1, H))

    agg2 = _agg_kernel(hn2_a, hn2_b, er_agg)
    h2 = _conv_call(False, False, agg2, pdeg_in, pdeg_out, conv2_W,
                    conv2_b.reshape(1, H), ln2_g.reshape(1, H),
                    ln2_b.reshape(1, H))

    hg_sum, hg_max = _pool_kernel(h2, offsets.reshape(128))

    return _head_call(hg_sum, hg_max, cnt, cls1_W, cls1_b.reshape(1, H),
                      cls2_W, cls2_b.reshape(1, H), cls3_W,
                      cls3_b.reshape(1, C), ln3_g.reshape(1, H),
                      ln3_b.reshape(1, H), ln4_g.reshape(1, H),
                      ln4_b.reshape(1, H))


# speedup vs baseline: 1.7398x; 1.0596x over previous
import functools

import jax
import jax.numpy as jnp
from jax import lax
from jax.experimental import pallas as pl
from jax.experimental.pallas import tpu as pltpu
from jax.experimental.pallas import tpu_sc as plsc

N, E, D, H, C, G = 10000, 160000, 256, 256, 16, 64
NP = 10240
NC, NS = 2, 16
NT = NC * NS
EPT = E // NT
EPS = E // NS
GPT = 16
GPN = NP // GPT
RPT = NP // NS
E3 = 163840
DCH = E3 // NT // 128
ACH = E3 // NS // 128

_MESH = plsc.VectorSubcoreMesh(core_axis_name="c", subcore_axis_name="s")


def _degree_body(edges, gids, out_o, out_i, out_c,
                 acc_o, acc_i, acc_c, sb0, sb1, sb2, sb3, db0, db1, db2, db3,
                 gid_a, ones_v, zb, sem):
    c = lax.axis_index("c")
    s = lax.axis_index("s")
    w = s * NC + c

    @pl.when(w < GPT)
    def _():
        pltpu.sync_copy(gids.at[w], gid_a)

    def _fill(i, _):
        zb[pl.ds(i * 16, 16)] = jnp.zeros((16,), jnp.float32)
        return 0
    lax.fori_loop(0, RPT // 16, _fill, 0)
    for i in range(8):
        ones_v[pl.ds(i * 16, 16)] = jnp.ones((16,), jnp.float32)

    pltpu.sync_copy(zb, acc_o.at[pl.ds(s * RPT, RPT)])
    pltpu.sync_copy(zb, acc_i.at[pl.ds(s * RPT, RPT)])

    @pl.when(s == 0)
    def _():
        pltpu.sync_copy(zb.at[pl.ds(0, 128)], acc_c)

    plsc.subcore_barrier()

    sbufs = (sb0, sb1, sb2, sb3)
    dbufs = (db0, db1, db2, db3)

    def _grp(gi, _):
        for k in range(4):
            j = gi * 4 + k
            pltpu.sync_copy(edges.at[0, w, j], sbufs[k])
            pltpu.sync_copy(edges.at[1, w, j], dbufs[k])
        for k in range(4):
            pltpu.async_copy(ones_v, acc_o.at[sbufs[k]], sem, add=True)
            pltpu.async_copy(ones_v, acc_i.at[dbufs[k]], sem, add=True)
        for k in range(4):
            pltpu.make_async_copy(ones_v, acc_o.at[sbufs[0]], sem).wait()
            pltpu.make_async_copy(ones_v, acc_i.at[dbufs[0]], sem).wait()
        return 0
    lax.fori_loop(0, DCH // 4, _grp, 0)

    @pl.when(w < GPT)
    def _():
        def _gfire(j, _):
            pltpu.async_copy(ones_v, acc_c.at[gid_a.at[j]], sem, add=True)
            return 0
        lax.fori_loop(0, GPN // 128, _gfire, 0)

        def _gdrain(j, _):
            pltpu.make_async_copy(ones_v, acc_c.at[gid_a.at[0]], sem).wait()
            return 0
        lax.fori_loop(0, GPN // 128, _gdrain, 0)

    plsc.subcore_barrier()
    pltpu.sync_copy(acc_o.at[pl.ds(s * RPT, RPT)], out_o.at[c, pl.ds(s * RPT, RPT)])
    pltpu.sync_copy(acc_i.at[pl.ds(s * RPT, RPT)], out_i.at[c, pl.ds(s * RPT, RPT)])

    @pl.when(s == 0)
    def _():
        pltpu.sync_copy(acc_c, out_c.at[c])


_degree_kernel = pl.kernel(
    _degree_body,
    out_type=(jax.ShapeDtypeStruct((NC, NP), jnp.float32),
              jax.ShapeDtypeStruct((NC, NP), jnp.float32),
              jax.ShapeDtypeStruct((NC, 128), jnp.float32)),
    mesh=_MESH,
    scratch_types=[
        pltpu.VMEM_SHARED((NP,), jnp.float32),
        pltpu.VMEM_SHARED((NP,), jnp.float32),
        pltpu.VMEM_SHARED((128,), jnp.float32),
        pltpu.VMEM((128,), jnp.int32),
        pltpu.VMEM((128,), jnp.int32),
        pltpu.VMEM((128,), jnp.int32),
        pltpu.VMEM((128,), jnp.int32),
        pltpu.VMEM((128,), jnp.int32),
        pltpu.VMEM((128,), jnp.int32),
        pltpu.VMEM((128,), jnp.int32),
        pltpu.VMEM((128,), jnp.int32),
        pltpu.VMEM((GPN // 128, 128), jnp.int32),
        pltpu.VMEM((128,), jnp.float32),
        pltpu.VMEM((RPT,), jnp.float32),
        pltpu.SemaphoreType.DMA,
    ],
)


def _agg_body(hn_a, hn_b, edges, out, acc, src_v, src_b, dst_v, dst_b,
              src16, dst16, rows, sem, se1a, se1b, se2a, se2b):
    se1 = (se1a, se1b)
    se2 = (se2a, se2b)
    c = lax.axis_index("c")
    s = lax.axis_index("s")

    def _zfill(i, _):
        for q in range(8):
            rows[i, pl.ds(q * 16, 16)] = jnp.zeros((16,), jnp.float32)
        return 0
    lax.fori_loop(0, 128, _zfill, 0)
    for t in range(RPT // 128):
        pltpu.sync_copy(rows, acc.at[pl.ds(s * RPT + t * 128, 128)])
    plsc.subcore_barrier()

    def _run(hn):
        sv = (src_v, src_b)
        dv = (dst_v, dst_b)

        def _pair(jj, _):
            ds_ = []
            for u in range(2):
                j = 2 * jj + u
                ds_.append(pltpu.async_copy(
                    edges.at[0, s, pl.ds(j * 128, 128)], sv[u], se1[u]))
                ds_.append(pltpu.async_copy(
                    edges.at[1, s, pl.ds(j * 128, 128)], dv[u], se2[u]))
            for u in range(2):
                ds_[2 * u].wait()
                dg = pltpu.async_copy(hn.at[sv[u]], rows, sem)
                ds_[2 * u + 1].wait()
                dg.wait()
                pltpu.sync_copy(rows, acc.at[dv[u]], add=True)
            return 0
        lax.fori_loop(0, EPS // 256, _pair, 0)
        tail = (EPS // 256) * 256
        pltpu.sync_copy(edges.at[0, s, pl.ds(tail, 16)], src16)
        pltpu.sync_copy(edges.at[1, s, pl.ds(tail, 16)], dst16)
        pltpu.async_copy(hn.at[src16], rows.at[pl.ds(0, 16)], sem).wait()
        pltpu.sync_copy(rows.at[pl.ds(0, 16)], acc.at[dst16], add=True)

    @pl.when(c == 0)
    def _():
        _run(hn_a)

    @pl.when(c == 1)
    def _():
        _run(hn_b)

    plsc.subcore_barrier()
    for t in range(RPT // 128):
        pltpu.sync_copy(acc.at[pl.ds(s * RPT + t * 128, 128)],
                        out.at[c, pl.ds(s * RPT + t * 128, 128)])


_agg_kernel = pl.kernel(
    _agg_body,
    out_type=jax.ShapeDtypeStruct((NC, NP, 128), jnp.float32),
    mesh=_MESH,
    scratch_types=[
        pltpu.VMEM_SHARED((NP, 128), jnp.float32),
        pltpu.VMEM((128,), jnp.int32),
        pltpu.VMEM((128,), jnp.int32),
        pltpu.VMEM((128,), jnp.int32),
        pltpu.VMEM((128,), jnp.int32),
        pltpu.VMEM((16,), jnp.int32),
        pltpu.VMEM((16,), jnp.int32),
        pltpu.VMEM((128, 128), jnp.float32),
        pltpu.SemaphoreType.DMA,
        pltpu.SemaphoreType.DMA,
        pltpu.SemaphoreType.DMA,
        pltpu.SemaphoreType.DMA,
        pltpu.SemaphoreType.DMA,
    ],
)


def _pool_body(h2, offs, out_s, out_m, off_v, rb, ob):
    c = lax.axis_index("c")
    s = lax.axis_index("s")
    w = s * NC + c
    pltpu.sync_copy(offs, off_v)

    for gi in range(2):
        g = w * 2 + gi
        ov = off_v[pl.ds(g, 16)]
        start = ov[0]
        end = ov[1]
        start_al = lax.div(start, 8) * 8
        nch = (end - start_al + 15) // 16

        def _chunk(k, carry):
            r0 = start_al + k * 16
            pltpu.sync_copy(h2.at[pl.ds(r0, 16)], rb)
            sums = list(carry[:16])
            maxs = list(carry[16:])
            for r in range(16):
                valid = ((r0 + r) >= start) & ((r0 + r) < end)
                for q in range(16):
                    v = rb[r, pl.ds(q * 16, 16)]
                    sums[q] = sums[q] + jnp.where(valid, v, 0.0)
                    maxs[q] = jnp.maximum(maxs[q], jnp.where(valid, v, -jnp.inf))
            return tuple(sums + maxs)

        init = tuple([jnp.zeros((16,), jnp.float32)] * 16
                     + [jnp.full((16,), -jnp.inf, jnp.float32)] * 16)
        carry = lax.fori_loop(0, nch, _chunk, init)
        for q in range(16):
            ob[pl.ds(q * 16, 16)] = carry[q]
        pltpu.sync_copy(ob, out_s.at[g])
        for q in range(16):
            ob[pl.ds(q * 16, 16)] = carry[16 + q]
        pltpu.sync_copy(ob, out_m.at[g])


_pool_kernel = pl.kernel(
    _pool_body,
    out_type=(jax.ShapeDtypeStruct((G, H), jnp.float32),
              jax.ShapeDtypeStruct((G, H), jnp.float32)),
    mesh=_MESH,
    scratch_types=[
        pltpu.VMEM((128,), jnp.int32),
        pltpu.VMEM((16, H), jnp.float32),
        pltpu.VMEM((H,), jnp.float32),
    ],
)


_B2 = 400


def _scale_body(x_ref, po_ref, pc_ref, hna_ref, hnb_ref, off_ref, cnt_ref):
    deg = po_ref[0] + po_ref[1]
    ns = lax.rsqrt(jnp.maximum(deg, 1.0))
    xb = x_ref[...]
    hna_ref[...] = xb[:, :128] * ns
    hnb_ref[...] = xb[:, 128:] * ns
    cnt = pc_ref[0] + pc_ref[1]
    cnt_ref[...] = cnt
    row = lax.broadcasted_iota(jnp.int32, (128, 128), 0)
    col = lax.broadcasted_iota(jnp.int32, (128, 128), 1)
    m = (row < col).astype(jnp.float32)
    off = jnp.dot(cnt, m, preferred_element_type=jnp.float32)
    off_ref[...] = off.astype(jnp.int32)


def _scale_call(x, pdeg_out, pcnt):
    return pl.pallas_call(
        _scale_body,
        grid=(25,),
        in_specs=[
            pl.BlockSpec((_B2, 256), lambda i: (i, 0)),
            pl.BlockSpec((NC, _B2, 1), lambda i: (0, i, 0)),
            pl.BlockSpec((NC, 1, 128), lambda i: (0, 0, 0)),
        ],
        out_specs=[
            pl.BlockSpec((_B2, 128), lambda i: (i, 0)),
            pl.BlockSpec((_B2, 128), lambda i: (i, 0)),
            pl.BlockSpec((1, 128), lambda i: (0, 0)),
            pl.BlockSpec((1, 128), lambda i: (0, 0)),
        ],
        out_shape=(jax.ShapeDtypeStruct((N, 128), jnp.float32),
                   jax.ShapeDtypeStruct((N, 128), jnp.float32),
                   jax.ShapeDtypeStruct((1, 128), jnp.int32),
                   jax.ShapeDtypeStruct((1, 128), jnp.float32)),
    )(x, pdeg_out, pcnt)


_BC = 512


def _conv_body(split_out, scale_src, a_ref, di_ref, do_ref, w_ref, b_ref,
               g_ref, be_ref, *out_refs):
    di = di_ref[0] + di_ref[1]
    nd = lax.rsqrt(jnp.maximum(di, 1.0))
    a0 = a_ref[0] * nd
    a1 = a_ref[1] * nd
    z = (jnp.dot(a0, w_ref[:128, :], preferred_element_type=jnp.float32)
         + jnp.dot(a1, w_ref[128:, :], preferred_element_type=jnp.float32)
         + b_ref[...])
    mu = jnp.mean(z, axis=-1, keepdims=True)
    var = jnp.mean((z - mu) ** 2, axis=-1, keepdims=True)
    h = (z - mu) / jnp.sqrt(var + 1e-5) * g_ref[...] + be_ref[...]
    h = jnp.maximum(h, 0.0)
    if scale_src:
        do = do_ref[0] + do_ref[1]
        h = h * lax.rsqrt(jnp.maximum(do, 1.0))
    if split_out:
        out_refs[0][...] = h[:, :128]
        out_refs[1][...] = h[:, 128:]
    else:
        out_refs[0][...] = h


def _conv_call(split_out, scale_src, agg, pdeg_in, pdeg_out, W, b, g, be):
    if split_out:
        out_specs = [pl.BlockSpec((_BC, 128), lambda i: (i, 0)),
                     pl.BlockSpec((_BC, 128), lambda i: (i, 0))]
        out_shape = (jax.ShapeDtypeStruct((NP, 128), jnp.float32),
                     jax.ShapeDtypeStruct((NP, 128), jnp.float32))
    else:
        out_specs = pl.BlockSpec((_BC, H), lambda i: (i, 0))
        out_shape = jax.ShapeDtypeStruct((NP, H), jnp.float32)
    return pl.pallas_call(
        functools.partial(_conv_body, split_out, scale_src),
        grid=(NP // _BC,),
        in_specs=[
            pl.BlockSpec((NC, _BC, 128), lambda i: (0, i, 0)),
            pl.BlockSpec((NC, _BC, 1), lambda i: (0, i, 0)),
            pl.BlockSpec((NC, _BC, 1), lambda i: (0, i, 0)),
            pl.BlockSpec((H, H), lambda i: (0, 0)),
            pl.BlockSpec((1, H), lambda i: (0, 0)),
            pl.BlockSpec((1, H), lambda i: (0, 0)),
            pl.BlockSpec((1, H), lambda i: (0, 0)),
        ],
        out_specs=out_specs,
        out_shape=out_shape,
    )(agg, pdeg_in, pdeg_out, W, b, g, be)


def _head_body(hs_ref, hx_ref, cnt_ref, w1_ref, b1_ref, w2_ref, b2_ref,
               w3_ref, b3_ref, g3_ref, be3_ref, g4_ref, be4_ref, out_ref):
    row = lax.broadcasted_iota(jnp.int32, (G, 128), 0)
    col = lax.broadcasted_iota(jnp.int32, (G, 128), 1)
    m = (row == col).astype(jnp.float32)
    cc = lax.dot_general(m, cnt_ref[...], (((1,), (1,)), ((), ())),
                         preferred_element_type=jnp.float32)
    mean = hs_ref[...] / jnp.maximum(cc, 1.0)

    def _l2n(v):
        n = jnp.sqrt(jnp.sum(v * v, axis=-1, keepdims=True))
        return v / jnp.maximum(n, 1e-12)

    def _ln(z, gg, bb):
        mu = jnp.mean(z, axis=-1, keepdims=True)
        var = jnp.mean((z - mu) ** 2, axis=-1, keepdims=True)
        return (z - mu) / jnp.sqrt(var + 1e-5) * gg + bb

    hm = _l2n(mean)
    hx = _l2n(hx_ref[...])
    z = (jnp.dot(hm, w1_ref[:H, :], preferred_element_type=jnp.float32)
         + jnp.dot(hx, w1_ref[H:, :], preferred_element_type=jnp.float32)
         + b1_ref[...])
    z = jnp.maximum(_ln(z, g3_ref[...], be3_ref[...]), 0.0)
    z = jnp.dot(z, w2_ref[...], preferred_element_type=jnp.float32) + b2_ref[...]
    z = jnp.maximum(_ln(z, g4_ref[...], be4_ref[...]), 0.0)
    out_ref[...] = (jnp.dot(z, w3_ref[...], preferred_element_type=jnp.float32)
                    + b3_ref[...])


def _head_call(hg_sum, hg_max, cnt, w1, b1, w2, b2, w3, b3, g3, be3, g4, be4):
    return pl.pallas_call(
        _head_body,
        out_shape=jax.ShapeDtypeStruct((G, C), jnp.float32),
    )(hg_sum, hg_max, cnt, w1, b1, w2, b2, w3, b3, g3, be3, g4, be4)


def kernel(x, edge_index, graph_ids, conv1_W, conv1_b, conv2_W, conv2_b,
           ln1_g, ln1_b, ln2_g, ln2_b, ln3_g, ln3_b, ln4_g, ln4_b,
           cls1_W, cls1_b, cls2_W, cls2_b, cls3_W, cls3_b):
    ei = edge_index.astype(jnp.int32)
    pad_d = jnp.full((2, E3 - E), N, jnp.int32)
    er_deg = jnp.concatenate([ei, pad_d], axis=1).reshape(2, NT, DCH, 128)
    er_agg = ei.reshape(2, NS, EPS)
    gr = jnp.concatenate([graph_ids.astype(jnp.int32),
                          jnp.full((NP - N,), G, jnp.int32)]).reshape(
        GPT, GPN // 128, 128)

    pdeg_out, pdeg_in, pcnt = _degree_kernel(er_deg, gr)
    pdeg_out = pdeg_out.reshape(NC, NP, 1)
    pdeg_in = pdeg_in.reshape(NC, NP, 1)
    pcnt = pcnt.reshape(NC, 1, 128)

    hn_a, hn_b, offsets, cnt = _scale_call(x, pdeg_out, pcnt)

    agg1 = _agg_kernel(hn_a, hn_b, er_agg)
    hn2_a, hn2_b = _conv_call(True, True, agg1, pdeg_in, pdeg_out, conv1_W,
                              conv1_b.reshape(1, H), ln1_g.reshape(1, H),
                              ln1_b.reshape(1, H))

    agg2 = _agg_kernel(hn2_a, hn2_b, er_agg)
    h2 = _conv_call(False, False, agg2, pdeg_in, pdeg_out, conv2_W,
                    conv2_b.reshape(1, H), ln2_g.reshape(1, H),
                    ln2_b.reshape(1, H))

    hg_sum, hg_max = _pool_kernel(h2, offsets.reshape(128))

    return _head_call(hg_sum, hg_max, cnt, cls1_W, cls1_b.reshape(1, H),
                      cls2_W, cls2_b.reshape(1, H), cls3_W,
                      cls3_b.reshape(1, C), ln3_g.reshape(1, H),
                      ln3_b.reshape(1, H), ln4_g.reshape(1, H),
                      ln4_b.reshape(1, H))


# speedup vs baseline: 1.8398x; 1.0575x over previous
import functools

import jax
import jax.numpy as jnp
from jax import lax
from jax.experimental import pallas as pl
from jax.experimental.pallas import tpu as pltpu
from jax.experimental.pallas import tpu_sc as plsc

N, E, D, H, C, G = 10000, 160000, 256, 256, 16, 64
NP = 10240
NC, NS = 2, 16
NT = NC * NS
EPT = E // NT
EPS = E // NS
GPT = 16
GPN = NP // GPT
RPT = NP // NS
E3 = 163840
DCH = E3 // NT // 128
ACH = E3 // NS // 128

_MESH = plsc.VectorSubcoreMesh(core_axis_name="c", subcore_axis_name="s")


def _degree_body(edges, gids, out_o, out_i, out_c,
                 acc_o, acc_i, acc_c, sb0, sb1, sb2, sb3, db0, db1, db2, db3,
                 gid_a, ones_v, zb, sem, si0, si1, si2, si3,
                 di0, di1, di2, di3):
    si = (si0, si1, si2, si3)
    di = (di0, di1, di2, di3)
    c = lax.axis_index("c")
    s = lax.axis_index("s")
    w = s * NC + c

    @pl.when(w < GPT)
    def _():
        pltpu.sync_copy(gids.at[w], gid_a)

    def _fill(i, _):
        zb[pl.ds(i * 16, 16)] = jnp.zeros((16,), jnp.float32)
        return 0
    lax.fori_loop(0, RPT // 16, _fill, 0)
    for i in range(8):
        ones_v[pl.ds(i * 16, 16)] = jnp.ones((16,), jnp.float32)

    pltpu.sync_copy(zb, acc_o.at[pl.ds(s * RPT, RPT)])
    pltpu.sync_copy(zb, acc_i.at[pl.ds(s * RPT, RPT)])

    @pl.when(s == 0)
    def _():
        pltpu.sync_copy(zb.at[pl.ds(0, 128)], acc_c)

    plsc.subcore_barrier()

    sbufs = (sb0, sb1, sb2, sb3)
    dbufs = (db0, db1, db2, db3)

    def _grp(gi, _):
        ds_ = []
        for k in range(4):
            j = gi * 4 + k
            ds_.append(pltpu.async_copy(edges.at[0, w, j], sbufs[k], si[k]))
            ds_.append(pltpu.async_copy(edges.at[1, w, j], dbufs[k], di[k]))
        for k in range(4):
            ds_[2 * k].wait()
            pltpu.async_copy(ones_v, acc_o.at[sbufs[k]], sem, add=True)
            ds_[2 * k + 1].wait()
            pltpu.async_copy(ones_v, acc_i.at[dbufs[k]], sem, add=True)
        for k in range(4):
            pltpu.make_async_copy(ones_v, acc_o.at[sbufs[0]], sem).wait()
            pltpu.make_async_copy(ones_v, acc_i.at[dbufs[0]], sem).wait()
        return 0
    lax.fori_loop(0, DCH // 4, _grp, 0)

    @pl.when(w < GPT)
    def _():
        def _gfire(j, _):
            pltpu.async_copy(ones_v, acc_c.at[gid_a.at[j]], sem, add=True)
            return 0
        lax.fori_loop(0, GPN // 128, _gfire, 0)

        def _gdrain(j, _):
            pltpu.make_async_copy(ones_v, acc_c.at[gid_a.at[0]], sem).wait()
            return 0
        lax.fori_loop(0, GPN // 128, _gdrain, 0)

    plsc.subcore_barrier()
    pltpu.sync_copy(acc_o.at[pl.ds(s * RPT, RPT)], out_o.at[c, pl.ds(s * RPT, RPT)])
    pltpu.sync_copy(acc_i.at[pl.ds(s * RPT, RPT)], out_i.at[c, pl.ds(s * RPT, RPT)])

    @pl.when(s == 0)
    def _():
        pltpu.sync_copy(acc_c, out_c.at[c])


_degree_kernel = pl.kernel(
    _degree_body,
    out_type=(jax.ShapeDtypeStruct((NC, NP), jnp.float32),
              jax.ShapeDtypeStruct((NC, NP), jnp.float32),
              jax.ShapeDtypeStruct((NC, 128), jnp.float32)),
    mesh=_MESH,
    scratch_types=[
        pltpu.VMEM_SHARED((NP,), jnp.float32),
        pltpu.VMEM_SHARED((NP,), jnp.float32),
        pltpu.VMEM_SHARED((128,), jnp.float32),
        pltpu.VMEM((128,), jnp.int32),
        pltpu.VMEM((128,), jnp.int32),
        pltpu.VMEM((128,), jnp.int32),
        pltpu.VMEM((128,), jnp.int32),
        pltpu.VMEM((128,), jnp.int32),
        pltpu.VMEM((128,), jnp.int32),
        pltpu.VMEM((128,), jnp.int32),
        pltpu.VMEM((128,), jnp.int32),
        pltpu.VMEM((GPN // 128, 128), jnp.int32),
        pltpu.VMEM((128,), jnp.float32),
        pltpu.VMEM((RPT,), jnp.float32),
        pltpu.SemaphoreType.DMA,
        pltpu.SemaphoreType.DMA,
        pltpu.SemaphoreType.DMA,
        pltpu.SemaphoreType.DMA,
        pltpu.SemaphoreType.DMA,
        pltpu.SemaphoreType.DMA,
        pltpu.SemaphoreType.DMA,
        pltpu.SemaphoreType.DMA,
        pltpu.SemaphoreType.DMA,
    ],
)


def _agg_body(hn_a, hn_b, edges, out, acc, src_v, src_b, dst_v, dst_b,
              src16, dst16, rows, sem, se1a, se1b, se2a, se2b):
    se1 = (se1a, se1b)
    se2 = (se2a, se2b)
    c = lax.axis_index("c")
    s = lax.axis_index("s")

    def _zfill(i, _):
        for q in range(8):
            rows[i, pl.ds(q * 16, 16)] = jnp.zeros((16,), jnp.float32)
        return 0
    lax.fori_loop(0, 128, _zfill, 0)
    for t in range(RPT // 128):
        pltpu.sync_copy(rows, acc.at[pl.ds(s * RPT + t * 128, 128)])
    plsc.subcore_barrier()

    def _run(hn):
        sv = (src_v, src_b)
        dv = (dst_v, dst_b)

        def _pair(jj, _):
            ds_ = []
            for u in range(2):
                j = 2 * jj + u
                ds_.append(pltpu.async_copy(
                    edges.at[0, s, pl.ds(j * 128, 128)], sv[u], se1[u]))
                ds_.append(pltpu.async_copy(
                    edges.at[1, s, pl.ds(j * 128, 128)], dv[u], se2[u]))
            for u in range(2):
                ds_[2 * u].wait()
                dg = pltpu.async_copy(hn.at[sv[u]], rows, sem)
                ds_[2 * u + 1].wait()
                dg.wait()
                pltpu.sync_copy(rows, acc.at[dv[u]], add=True)
            return 0
        lax.fori_loop(0, EPS // 256, _pair, 0)
        tail = (EPS // 256) * 256
        pltpu.sync_copy(edges.at[0, s, pl.ds(tail, 16)], src16)
        pltpu.sync_copy(edges.at[1, s, pl.ds(tail, 16)], dst16)
        pltpu.async_copy(hn.at[src16], rows.at[pl.ds(0, 16)], sem).wait()
        pltpu.sync_copy(rows.at[pl.ds(0, 16)], acc.at[dst16], add=True)

    @pl.when(c == 0)
    def _():
        _run(hn_a)

    @pl.when(c == 1)
    def _():
        _run(hn_b)

    plsc.subcore_barrier()
    for t in range(RPT // 128):
        pltpu.sync_copy(acc.at[pl.ds(s * RPT + t * 128, 128)],
                        out.at[c, pl.ds(s * RPT + t * 128, 128)])


_agg_kernel = pl.kernel(
    _agg_body,
    out_type=jax.ShapeDtypeStruct((NC, NP, 128), jnp.float32),
    mesh=_MESH,
    scratch_types=[
        pltpu.VMEM_SHARED((NP, 128), jnp.float32),
        pltpu.VMEM((128,), jnp.int32),
        pltpu.VMEM((128,), jnp.int32),
        pltpu.VMEM((128,), jnp.int32),
        pltpu.VMEM((128,), jnp.int32),
        pltpu.VMEM((16,), jnp.int32),
        pltpu.VMEM((16,), jnp.int32),
        pltpu.VMEM((128, 128), jnp.float32),
        pltpu.SemaphoreType.DMA,
        pltpu.SemaphoreType.DMA,
        pltpu.SemaphoreType.DMA,
        pltpu.SemaphoreType.DMA,
        pltpu.SemaphoreType.DMA,
    ],
)


def _pool_body(h2, offs, out_s, out_m, off_v, rb, ob):
    c = lax.axis_index("c")
    s = lax.axis_index("s")
    w = s * NC + c
    pltpu.sync_copy(offs, off_v)

    for gi in range(2):
        g = w * 2 + gi
        ov = off_v[pl.ds(g, 16)]
        start = ov[0]
        end = ov[1]
        start_al = lax.div(start, 8) * 8
        nch = (end - start_al + 15) // 16

        def _chunk(k, carry):
            r0 = start_al + k * 16
            pltpu.sync_copy(h2.at[pl.ds(r0, 16)], rb)
            sums = list(carry[:16])
            maxs = list(carry[16:])
            for r in range(16):
                valid = ((r0 + r) >= start) & ((r0 + r) < end)
                for q in range(16):
                    v = rb[r, pl.ds(q * 16, 16)]
                    sums[q] = sums[q] + jnp.where(valid, v, 0.0)
                    maxs[q] = jnp.maximum(maxs[q], jnp.where(valid, v, -jnp.inf))
            return tuple(sums + maxs)

        init = tuple([jnp.zeros((16,), jnp.float32)] * 16
                     + [jnp.full((16,), -jnp.inf, jnp.float32)] * 16)
        carry = lax.fori_loop(0, nch, _chunk, init)
        for q in range(16):
            ob[pl.ds(q * 16, 16)] = carry[q]
        pltpu.sync_copy(ob, out_s.at[g])
        for q in range(16):
            ob[pl.ds(q * 16, 16)] = carry[16 + q]
        pltpu.sync_copy(ob, out_m.at[g])


_pool_kernel = pl.kernel(
    _pool_body,
    out_type=(jax.ShapeDtypeStruct((G, H), jnp.float32),
              jax.ShapeDtypeStruct((G, H), jnp.float32)),
    mesh=_MESH,
    scratch_types=[
        pltpu.VMEM((128,), jnp.int32),
        pltpu.VMEM((16, H), jnp.float32),
        pltpu.VMEM((H,), jnp.float32),
    ],
)


_B2 = 400


def _scale_body(x_ref, po_ref, pc_ref, hna_ref, hnb_ref, off_ref, cnt_ref):
    deg = po_ref[0] + po_ref[1]
    ns = lax.rsqrt(jnp.maximum(deg, 1.0))
    xb = x_ref[...]
    hna_ref[...] = xb[:, :128] * ns
    hnb_ref[...] = xb[:, 128:] * ns
    cnt = pc_ref[0] + pc_ref[1]
    cnt_ref[...] = cnt
    row = lax.broadcasted_iota(jnp.int32, (128, 128), 0)
    col = lax.broadcasted_iota(jnp.int32, (128, 128), 1)
    m = (row < col).astype(jnp.float32)
    off = jnp.dot(cnt, m, preferred_element_type=jnp.float32)
    off_ref[...] = off.astype(jnp.int32)


def _scale_call(x, pdeg_out, pcnt):
    return pl.pallas_call(
        _scale_body,
        grid=(25,),
        in_specs=[
            pl.BlockSpec((_B2, 256), lambda i: (i, 0)),
            pl.BlockSpec((NC, _B2, 1), lambda i: (0, i, 0)),
            pl.BlockSpec((NC, 1, 128), lambda i: (0, 0, 0)),
        ],
        out_specs=[
            pl.BlockSpec((_B2, 128), lambda i: (i, 0)),
            pl.BlockSpec((_B2, 128), lambda i: (i, 0)),
            pl.BlockSpec((1, 128), lambda i: (0, 0)),
            pl.BlockSpec((1, 128), lambda i: (0, 0)),
        ],
        out_shape=(jax.ShapeDtypeStruct((N, 128), jnp.float32),
                   jax.ShapeDtypeStruct((N, 128), jnp.float32),
                   jax.ShapeDtypeStruct((1, 128), jnp.int32),
                   jax.ShapeDtypeStruct((1, 128), jnp.float32)),
    )(x, pdeg_out, pcnt)


_BC = 512


def _conv_body(split_out, scale_src, a_ref, di_ref, do_ref, w_ref, b_ref,
               g_ref, be_ref, *out_refs):
    di = di_ref[0] + di_ref[1]
    nd = lax.rsqrt(jnp.maximum(di, 1.0))
    a0 = a_ref[0] * nd
    a1 = a_ref[1] * nd
    z = (jnp.dot(a0, w_ref[:128, :], preferred_element_type=jnp.float32)
         + jnp.dot(a1, w_ref[128:, :], preferred_element_type=jnp.float32)
         + b_ref[...])
    mu = jnp.mean(z, axis=-1, keepdims=True)
    var = jnp.mean((z - mu) ** 2, axis=-1, keepdims=True)
    h = (z - mu) / jnp.sqrt(var + 1e-5) * g_ref[...] + be_ref[...]
    h = jnp.maximum(h, 0.0)
    if scale_src:
        do = do_ref[0] + do_ref[1]
        h = h * lax.rsqrt(jnp.maximum(do, 1.0))
    if split_out:
        out_refs[0][...] = h[:, :128]
        out_refs[1][...] = h[:, 128:]
    else:
        out_refs[0][...] = h


def _conv_call(split_out, scale_src, agg, pdeg_in, pdeg_out, W, b, g, be):
    if split_out:
        out_specs = [pl.BlockSpec((_BC, 128), lambda i: (i, 0)),
                     pl.BlockSpec((_BC, 128), lambda i: (i, 0))]
        out_shape = (jax.ShapeDtypeStruct((NP, 128), jnp.float32),
                     jax.ShapeDtypeStruct((NP, 128), jnp.float32))
    else:
        out_specs = pl.BlockSpec((_BC, H), lambda i: (i, 0))
        out_shape = jax.ShapeDtypeStruct((NP, H), jnp.float32)
    return pl.pallas_call(
        functools.partial(_conv_body, split_out, scale_src),
        grid=(NP // _BC,),
        in_specs=[
            pl.BlockSpec((NC, _BC, 128), lambda i: (0, i, 0)),
            pl.BlockSpec((NC, _BC, 1), lambda i: (0, i, 0)),
            pl.BlockSpec((NC, _BC, 1), lambda i: (0, i, 0)),
            pl.BlockSpec((H, H), lambda i: (0, 0)),
            pl.BlockSpec((1, H), lambda i: (0, 0)),
            pl.BlockSpec((1, H), lambda i: (0, 0)),
            pl.BlockSpec((1, H), lambda i: (0, 0)),
        ],
        out_specs=out_specs,
        out_shape=out_shape,
    )(agg, pdeg_in, pdeg_out, W, b, g, be)


def _head_body(hs_ref, hx_ref, cnt_ref, w1_ref, b1_ref, w2_ref, b2_ref,
               w3_ref, b3_ref, g3_ref, be3_ref, g4_ref, be4_ref, out_ref):
    row = lax.broadcasted_iota(jnp.int32, (G, 128), 0)
    col = lax.broadcasted_iota(jnp.int32, (G, 128), 1)
    m = (row == col).astype(jnp.float32)
    cc = lax.dot_general(m, cnt_ref[...], (((1,), (1,)), ((), ())),
                         preferred_element_type=jnp.float32)
    mean = hs_ref[...] / jnp.maximum(cc, 1.0)

    def _l2n(v):
        n = jnp.sqrt(jnp.sum(v * v, axis=-1, keepdims=True))
        return v / jnp.maximum(n, 1e-12)

    def _ln(z, gg, bb):
        mu = jnp.mean(z, axis=-1, keepdims=True)
        var = jnp.mean((z - mu) ** 2, axis=-1, keepdims=True)
        return (z - mu) / jnp.sqrt(var + 1e-5) * gg + bb

    hm = _l2n(mean)
    hx = _l2n(hx_ref[...])
    z = (jnp.dot(hm, w1_ref[:H, :], preferred_element_type=jnp.float32)
         + jnp.dot(hx, w1_ref[H:, :], preferred_element_type=jnp.float32)
         + b1_ref[...])
    z = jnp.maximum(_ln(z, g3_ref[...], be3_ref[...]), 0.0)
    z = jnp.dot(z, w2_ref[...], preferred_element_type=jnp.float32) + b2_ref[...]
    z = jnp.maximum(_ln(z, g4_ref[...], be4_ref[...]), 0.0)
    out_ref[...] = (jnp.dot(z, w3_ref[...], preferred_element_type=jnp.float32)
                    + b3_ref[...])


def _head_call(hg_sum, hg_max, cnt, w1, b1, w2, b2, w3, b3, g3, be3, g4, be4):
    return pl.pallas_call(
        _head_body,
        out_shape=jax.ShapeDtypeStruct((G, C), jnp.float32),
    )(hg_sum, hg_max, cnt, w1, b1, w2, b2, w3, b3, g3, be3, g4, be4)


def kernel(x, edge_index, graph_ids, conv1_W, conv1_b, conv2_W, conv2_b,
           ln1_g, ln1_b, ln2_g, ln2_b, ln3_g, ln3_b, ln4_g, ln4_b,
           cls1_W, cls1_b, cls2_W, cls2_b, cls3_W, cls3_b):
    ei = edge_index.astype(jnp.int32)
    pad_d = jnp.full((2, E3 - E), N, jnp.int32)
    er_deg = jnp.concatenate([ei, pad_d], axis=1).reshape(2, NT, DCH, 128)
    er_agg = ei.reshape(2, NS, EPS)
    gr = jnp.concatenate([graph_ids.astype(jnp.int32),
                          jnp.full((NP - N,), G, jnp.int32)]).reshape(
        GPT, GPN // 128, 128)

    pdeg_out, pdeg_in, pcnt = _degree_kernel(er_deg, gr)
    pdeg_out = pdeg_out.reshape(NC, NP, 1)
    pdeg_in = pdeg_in.reshape(NC, NP, 1)
    pcnt = pcnt.reshape(NC, 1, 128)

    hn_a, hn_b, offsets, cnt = _scale_call(x, pdeg_out, pcnt)

    agg1 = _agg_kernel(hn_a, hn_b, er_agg)
    hn2_a, hn2_b = _conv_call(True, True, agg1, pdeg_in, pdeg_out, conv1_W,
                              conv1_b.reshape(1, H), ln1_g.reshape(1, H),
                              ln1_b.reshape(1, H))

    agg2 = _agg_kernel(hn2_a, hn2_b, er_agg)
    h2 = _conv_call(False, False, agg2, pdeg_in, pdeg_out, conv2_W,
                    conv2_b.reshape(1, H), ln2_g.reshape(1, H),
                    ln2_b.reshape(1, H))

    hg_sum, hg_max = _pool_kernel(h2, offsets.reshape(128))

    return _head_call(hg_sum, hg_max, cnt, cls1_W, cls1_b.reshape(1, H),
                      cls2_W, cls2_b.reshape(1, H), cls3_W,
                      cls3_b.reshape(1, C), ln3_g.reshape(1, H),
                      ln3_b.reshape(1, H), ln4_g.reshape(1, H),
                      ln4_b.reshape(1, H))
